# Initial kernel scaffold; baseline (speedup 1.0000x reference)
import jax
import jax.numpy as jnp
from jax.experimental import pallas as pl


def kernel(atom_fea, edge, rbf, nbr_fea_idx, crystal_atom_idx, crystal_edge_idx, W_rbf, W_full, W_mask, gamma1, beta1, gamma2, beta2, res_W1_0, res_b1_0, res_W2_0, res_b2_0, res_W1_1, res_b1_1, res_W2_1, res_b2_1):
    raise NotImplementedError("write your pallas kernel here")


# speedup vs baseline: 4.7307x; 4.7307x over previous
import functools

import jax
import jax.numpy as jnp
from jax import lax
from jax.experimental import pallas as pl
from jax.experimental.pallas import tpu as pltpu
from jax.experimental.pallas import tpu_sc as plsc

_N = 10000
_E = 320000
_AF = 128
_NF = 16
_NR = 16
_NC = 256
_EPS = 1e-5

_NW = 32
_PER_W = _E // _NW
_GCH = 80
_SCH = 80
_ROWS_T = _N // 16

_BE = 2000
_NB = _E // _BE


def _sc_gather(atom_fea, idx_dst, idx_src):
    mesh = plsc.VectorSubcoreMesh(core_axis_name="c", subcore_axis_name="s")

    @functools.partial(
        pl.kernel,
        out_type=(
            jax.ShapeDtypeStruct((_E, _AF), jnp.float32),
            jax.ShapeDtypeStruct((_E, _AF), jnp.float32),
        ),
        mesh=mesh,
        scratch_types=[
            pltpu.VMEM((_GCH,), jnp.int32),
            pltpu.VMEM((_GCH,), jnp.int32),
            pltpu.VMEM((_GCH, _AF), jnp.float32),
            pltpu.VMEM((_GCH, _AF), jnp.float32),
            pltpu.SemaphoreType.DMA,
            pltpu.SemaphoreType.DMA,
        ],
    )
    def k(atom_hbm, i1_hbm, i2_hbm, o1_hbm, o2_hbm, i1_v, i2_v, b1, b2, s1, s2):
        wid = lax.axis_index("s") * 2 + lax.axis_index("c")
        base0 = wid * _PER_W

        def body(i, carry):
            base = base0 + i * _GCH
            pltpu.sync_copy(i1_hbm.at[pl.ds(base, _GCH)], i1_v)
            pltpu.sync_copy(i2_hbm.at[pl.ds(base, _GCH)], i2_v)
            c1 = pltpu.async_copy(atom_hbm.at[i1_v], b1, s1)
            c2 = pltpu.async_copy(atom_hbm.at[i2_v], b2, s2)
            c1.wait()
            c2.wait()
            pltpu.sync_copy(b1, o1_hbm.at[pl.ds(base, _GCH)])
            pltpu.sync_copy(b2, o2_hbm.at[pl.ds(base, _GCH)])
            return carry

        lax.fori_loop(0, _PER_W // _GCH, body, 0)

    return k(atom_fea, idx_dst, idx_src)


def _sc_scatter(msg, idx_dst, z128):
    mesh = plsc.VectorSubcoreMesh(core_axis_name="c", subcore_axis_name="s")

    @functools.partial(
        pl.kernel,
        out_type=jax.ShapeDtypeStruct((2, _N, _AF), jnp.float32),
        mesh=mesh,
        scratch_types=[
            pltpu.VMEM((_SCH,), jnp.int32),
            pltpu.VMEM((_SCH, _AF), jnp.float32),
            pltpu.VMEM_SHARED((_N, _AF), jnp.float32),
        ],
    )
    def k(msg_hbm, idx_hbm, z128_hbm, acc_hbm, idx_v, msg_v, acc_sh):
        cid = lax.axis_index("c")
        sid = lax.axis_index("s")
        wid = sid * 2 + cid
        base0 = wid * _PER_W

        @pl.when(sid == 0)
        def _():
            pltpu.sync_copy(z128_hbm, acc_sh)

        plsc.subcore_barrier()

        def body(i, carry):
            base = base0 + i * _SCH
            pltpu.sync_copy(idx_hbm.at[pl.ds(base, _SCH)], idx_v)
            pltpu.sync_copy(msg_hbm.at[pl.ds(base, _SCH)], msg_v)
            pltpu.sync_copy(msg_v, acc_sh.at[idx_v], add=True)
            return carry

        lax.fori_loop(0, _PER_W // _SCH, body, 0)
        plsc.subcore_barrier()

        r0 = pl.multiple_of(sid * 640, 8)

        @pl.when(sid < 15)
        def _():
            pltpu.sync_copy(acc_sh.at[pl.ds(r0, 640)],
                            acc_hbm.at[cid, pl.ds(r0, 640)])

        @pl.when(sid == 15)
        def _():
            pltpu.sync_copy(acc_sh.at[pl.ds(9600, 400)],
                            acc_hbm.at[cid, pl.ds(9600, 400)])

    return k(msg, idx_dst, z128)


def _sc_count(idx_dst, z128):
    mesh = plsc.VectorSubcoreMesh(core_axis_name="c", subcore_axis_name="s")

    @functools.partial(
        pl.kernel,
        out_type=jax.ShapeDtypeStruct((2, _N, _AF), jnp.float32),
        mesh=mesh,
        scratch_types=[
            pltpu.VMEM((_SCH,), jnp.int32),
            pltpu.VMEM((_SCH, _AF), jnp.float32),
            pltpu.VMEM_SHARED((_N, _AF), jnp.float32),
        ],
    )
    def k(idx_hbm, z128_hbm, cnt_hbm, idx_v, ones_v, cnt_sh):
        cid = lax.axis_index("c")
        sid = lax.axis_index("s")
        wid = sid * 2 + cid
        base0 = wid * _PER_W

        @pl.when(sid == 0)
        def _():
            pltpu.sync_copy(z128_hbm, cnt_sh)

        def initones(r, carry):
            ones_v[r, pl.ds(0, 16)] = jnp.ones((16,), jnp.float32)
            ones_v[r, pl.ds(16, 16)] = jnp.ones((16,), jnp.float32)
            ones_v[r, pl.ds(32, 16)] = jnp.ones((16,), jnp.float32)
            ones_v[r, pl.ds(48, 16)] = jnp.ones((16,), jnp.float32)
            ones_v[r, pl.ds(64, 16)] = jnp.ones((16,), jnp.float32)
            ones_v[r, pl.ds(80, 16)] = jnp.ones((16,), jnp.float32)
            ones_v[r, pl.ds(96, 16)] = jnp.ones((16,), jnp.float32)
            ones_v[r, pl.ds(112, 16)] = jnp.ones((16,), jnp.float32)
            return carry

        lax.fori_loop(0, _SCH, initones, 0)
        plsc.subcore_barrier()

        def body(i, carry):
            base = base0 + i * _SCH
            pltpu.sync_copy(idx_hbm.at[pl.ds(base, _SCH)], idx_v)
            pltpu.sync_copy(ones_v, cnt_sh.at[idx_v], add=True)
            return carry

        lax.fori_loop(0, _PER_W // _SCH, body, 0)
        plsc.subcore_barrier()

        r0 = pl.multiple_of(sid * 640, 8)

        @pl.when(sid < 15)
        def _():
            pltpu.sync_copy(cnt_sh.at[pl.ds(r0, 640)],
                            cnt_hbm.at[cid, pl.ds(r0, 640)])

        @pl.when(sid == 15)
        def _():
            pltpu.sync_copy(cnt_sh.at[pl.ds(9600, 400)],
                            cnt_hbm.at[cid, pl.ds(9600, 400)])

    return k(idx_dst, z128)


def _p1_body(a1_ref, a2_ref, edge_ref, rbf_ref, ce_ref, wrbf_ref, wfull_ref,
             tg_ref, sums_ref, cnt_ref):
    @pl.when(pl.program_id(0) == 0)
    def _():
        sums_ref[...] = jnp.zeros_like(sums_ref)
        cnt_ref[...] = jnp.zeros_like(cnt_ref)

    wa = wfull_ref[0:_AF, :]
    wb = wfull_ref[_AF:2 * _AF, :]
    wc = wfull_ref[2 * _AF:, :]
    nbr = edge_ref[...] * jnp.dot(rbf_ref[...], wrbf_ref[...],
                                  preferred_element_type=jnp.float32)
    tg = (jnp.dot(a1_ref[...], wa, preferred_element_type=jnp.float32)
          + jnp.dot(a2_ref[...], wb, preferred_element_type=jnp.float32)
          + jnp.dot(nbr, wc, preferred_element_type=jnp.float32))
    tg_ref[...] = tg

    ce = ce_ref[...]
    c_lo = jnp.min(ce)
    c_hi = jnp.max(ce)

    def crystal_iter(c, carry):
        m = (ce == c).astype(jnp.float32)
        mt = m * tg
        s_row = jnp.sum(mt, axis=0, keepdims=True)
        q_row = jnp.sum(mt * tg, axis=0, keepdims=True)
        n = jnp.sum(m)
        sums_ref[pl.ds(c, 1), 0:2 * _AF] += s_row
        sums_ref[pl.ds(c, 1), 2 * _AF:] += q_row
        cnt_ref[pl.ds(c, 1), :] += n
        return carry

    lax.fori_loop(c_lo, c_hi + 1, crystal_iter, 0)


def _edge_pass1(a1, a2, edge, rbf, ce2, W_rbf, W_full):
    return pl.pallas_call(
        _p1_body,
        grid=(_NB,),
        in_specs=[
            pl.BlockSpec((_BE, _AF), lambda i: (i, 0)),
            pl.BlockSpec((_BE, _AF), lambda i: (i, 0)),
            pl.BlockSpec((_BE, _NF), lambda i: (i, 0)),
            pl.BlockSpec((_BE, _NR), lambda i: (i, 0)),
            pl.BlockSpec((_BE, 1), lambda i: (i, 0)),
            pl.BlockSpec((_NR, _NF), lambda i: (0, 0)),
            pl.BlockSpec((2 * _AF + _NF, 2 * _AF), lambda i: (0, 0)),
        ],
        out_specs=[
            pl.BlockSpec((_BE, 2 * _AF), lambda i: (i, 0)),
            pl.BlockSpec((_NC, 4 * _AF), lambda i: (0, 0)),
            pl.BlockSpec((_NC, _AF), lambda i: (0, 0)),
        ],
        out_shape=[
            jax.ShapeDtypeStruct((_E, 2 * _AF), jnp.float32),
            jax.ShapeDtypeStruct((_NC, 4 * _AF), jnp.float32),
            jax.ShapeDtypeStruct((_NC, _AF), jnp.float32),
        ],
        compiler_params=pltpu.CompilerParams(
            dimension_semantics=("arbitrary",)),
    )(a1, a2, edge, rbf, ce2, W_rbf, W_full)


def _p2_body(tg_ref, ce_ref, sums_ref, cnt_ref, g1_ref, b1_ref, wm_ref,
             msg_ref):
    tg = tg_ref[...]
    ce = ce_ref[...]
    c_lo = jnp.min(ce)
    c_hi = jnp.max(ce)
    gamma = g1_ref[...]
    beta = b1_ref[...]

    def crystal_iter(c, carry):
        ae, be = carry
        n = jnp.maximum(cnt_ref[pl.ds(c, 1), 0:1], 1.0)
        srow = sums_ref[pl.ds(c, 1), 0:2 * _AF] / n
        qrow = sums_ref[pl.ds(c, 1), 2 * _AF:] / n
        var = jnp.maximum(qrow - srow * srow, 0.0)
        a = gamma * lax.rsqrt(var + _EPS)
        b = beta - srow * a
        m = (ce == c).astype(jnp.float32)
        return ae + m * a, be + m * b

    ae0 = jnp.zeros_like(tg)
    ae, be = lax.fori_loop(c_lo, c_hi + 1, crystal_iter, (ae0, ae0))
    tgn = tg * ae + be
    filt = jax.nn.sigmoid(jnp.dot(tgn[:, :_AF], wm_ref[...],
                                  preferred_element_type=jnp.float32))
    core = jnp.maximum(tgn[:, _AF:], 0.0)
    msg_ref[...] = filt * core


def _edge_pass2(tg, ce2, sums, cnt, gamma1, beta1, W_mask):
    return pl.pallas_call(
        _p2_body,
        grid=(_NB,),
        in_specs=[
            pl.BlockSpec((_BE, 2 * _AF), lambda i: (i, 0)),
            pl.BlockSpec((_BE, 1), lambda i: (i, 0)),
            pl.BlockSpec((_NC, 4 * _AF), lambda i: (0, 0)),
            pl.BlockSpec((_NC, _AF), lambda i: (0, 0)),
            pl.BlockSpec((1, 2 * _AF), lambda i: (0, 0)),
            pl.BlockSpec((1, 2 * _AF), lambda i: (0, 0)),
            pl.BlockSpec((_AF, 1), lambda i: (0, 0)),
        ],
        out_specs=pl.BlockSpec((_BE, _AF), lambda i: (i, 0)),
        out_shape=jax.ShapeDtypeStruct((_E, _AF), jnp.float32),
        compiler_params=pltpu.CompilerParams(
            dimension_semantics=("arbitrary",)),
    )(tg, ce2, sums, cnt, gamma1, beta1, W_mask)


def _fin_body(acc_ref, cnt_ref, af_ref, ca_ref, g2_ref, b2_ref,
              w10_ref, bb10_ref, w20_ref, bb20_ref,
              w11_ref, bb11_ref, w21_ref, bb21_ref, out_ref):
    acc = acc_ref[0] + acc_ref[1]
    cnt = cnt_ref[0, :, 0:1] + cnt_ref[1, :, 0:1]
    x = acc / jnp.maximum(cnt, 1.0)

    ca = ca_ref[...]
    iot = lax.broadcasted_iota(jnp.int32, (1, _NC), 1)
    onehot = (ca == iot).astype(jnp.float32)
    xcat = jnp.concatenate([x, x * x], axis=1)
    sums = lax.dot_general(onehot, xcat, (((0,), (0,)), ((), ())),
                           preferred_element_type=jnp.float32)
    cntc = jnp.maximum(
        jnp.sum(onehot, axis=0, keepdims=True), 1.0).reshape(_NC, 1)
    mean = sums[:, :_AF] / cntc
    var = jnp.maximum(sums[:, _AF:] / cntc - mean * mean, 0.0)
    ac = g2_ref[...] * lax.rsqrt(var + _EPS)
    bc = b2_ref[...] - mean * ac
    a_e = jnp.dot(onehot, ac, preferred_element_type=jnp.float32)
    b_e = jnp.dot(onehot, bc, preferred_element_type=jnp.float32)
    y = x * a_e + b_e

    h = jnp.maximum(jnp.dot(y, w10_ref[...],
                            preferred_element_type=jnp.float32)
                    + bb10_ref[...], 0.0)
    y = y + jnp.dot(h, w20_ref[...],
                    preferred_element_type=jnp.float32) + bb20_ref[...]
    h = jnp.maximum(jnp.dot(y, w11_ref[...],
                            preferred_element_type=jnp.float32)
                    + bb11_ref[...], 0.0)
    y = y + jnp.dot(h, w21_ref[...],
                    preferred_element_type=jnp.float32) + bb21_ref[...]

    out_ref[...] = 0.7071067811865476 * jnp.maximum(af_ref[...] + y, 0.0)


def _final(acc, cnt, atom_fea, ca2, gamma2, beta2,
           w10, b10, w20, b20, w11, b11, w21, b21):
    full = lambda s: pl.BlockSpec(s, lambda: tuple(0 for _ in s))
    return pl.pallas_call(
        _fin_body,
        in_specs=[
            full((2, _N, _AF)), full((2, _N, _AF)), full((_N, _AF)),
            full((_N, 1)), full((1, _AF)), full((1, _AF)),
            full((_AF, _AF // 2)), full((1, _AF // 2)),
            full((_AF // 2, _AF)), full((1, _AF)),
            full((_AF, _AF // 2)), full((1, _AF // 2)),
            full((_AF // 2, _AF)), full((1, _AF)),
        ],
        out_specs=full((_N, _AF)),
        out_shape=jax.ShapeDtypeStruct((_N, _AF), jnp.float32),
    )(acc, cnt, atom_fea, ca2, gamma2, beta2,
      w10, b10, w20, b20, w11, b11, w21, b21)


def kernel(atom_fea, edge, rbf, nbr_fea_idx, crystal_atom_idx,
           crystal_edge_idx, W_rbf, W_full, W_mask, gamma1, beta1, gamma2,
           beta2, res_W1_0, res_b1_0, res_W2_0, res_b2_0, res_W1_1,
           res_b1_1, res_W2_1, res_b2_1):
    idx_dst = nbr_fea_idx[:, 0]
    idx_src = nbr_fea_idx[:, 1]

    a1, a2 = _sc_gather(atom_fea, idx_dst, idx_src)

    ce2 = crystal_edge_idx.reshape(_E, 1)
    tg, sums, cnt = _edge_pass1(a1, a2, edge, rbf, ce2, W_rbf, W_full)
    msg = _edge_pass2(tg, ce2, sums, cnt, gamma1.reshape(1, -1),
                      beta1.reshape(1, -1), W_mask)

    z128 = jnp.zeros((_N, _AF), jnp.float32)
    acc = _sc_scatter(msg, idx_dst, z128)
    cnt_at = _sc_count(idx_dst, z128)

    out = _final(acc, cnt_at, atom_fea, crystal_atom_idx.reshape(_N, 1),
                 gamma2.reshape(1, -1), beta2.reshape(1, -1),
                 res_W1_0, res_b1_0.reshape(1, -1),
                 res_W2_0, res_b2_0.reshape(1, -1),
                 res_W1_1, res_b1_1.reshape(1, -1),
                 res_W2_1, res_b2_1.reshape(1, -1))
    return out


# speedup vs baseline: 5.6620x; 1.1969x over previous
import functools

import jax
import jax.numpy as jnp
from jax import lax
from jax.experimental import pallas as pl
from jax.experimental.pallas import tpu as pltpu
from jax.experimental.pallas import tpu_sc as plsc

_N = 10000
_E = 320000
_AF = 128
_NF = 16
_NR = 16
_NC = 256
_EPS = 1e-5

_NW = 32
_PER_W = _E // _NW
_GCH = 80
_SCH = 80
_ROWS_T = _N // 16

_BE = 2000
_NB = _E // _BE


def _sc_gather(atom_fea, idx_dst, idx_src):
    mesh = plsc.VectorSubcoreMesh(core_axis_name="c", subcore_axis_name="s")

    @functools.partial(
        pl.kernel,
        out_type=(
            jax.ShapeDtypeStruct((_E, _AF), jnp.float32),
            jax.ShapeDtypeStruct((_E, _AF), jnp.float32),
        ),
        mesh=mesh,
        scratch_types=[
            pltpu.VMEM((_GCH,), jnp.int32),
            pltpu.VMEM((_GCH,), jnp.int32),
            pltpu.VMEM((_GCH, _AF), jnp.float32),
            pltpu.VMEM((_GCH, _AF), jnp.float32),
            pltpu.SemaphoreType.DMA,
            pltpu.SemaphoreType.DMA,
        ],
    )
    def k(atom_hbm, i1_hbm, i2_hbm, o1_hbm, o2_hbm, i1_v, i2_v, b1, b2, s1, s2):
        wid = lax.axis_index("s") * 2 + lax.axis_index("c")
        base0 = wid * _PER_W

        def body(i, carry):
            base = base0 + i * _GCH
            pltpu.sync_copy(i1_hbm.at[pl.ds(base, _GCH)], i1_v)
            pltpu.sync_copy(i2_hbm.at[pl.ds(base, _GCH)], i2_v)
            c1 = pltpu.async_copy(atom_hbm.at[i1_v], b1, s1)
            c2 = pltpu.async_copy(atom_hbm.at[i2_v], b2, s2)
            c1.wait()
            c2.wait()
            pltpu.sync_copy(b1, o1_hbm.at[pl.ds(base, _GCH)])
            pltpu.sync_copy(b2, o2_hbm.at[pl.ds(base, _GCH)])
            return carry

        lax.fori_loop(0, _PER_W // _GCH, body, 0)

    return k(atom_fea, idx_dst, idx_src)


def _sc_scatter(msg, idx_dst, z128):
    mesh = plsc.VectorSubcoreMesh(core_axis_name="c", subcore_axis_name="s")

    @functools.partial(
        pl.kernel,
        out_type=jax.ShapeDtypeStruct((2, _N, _AF), jnp.float32),
        mesh=mesh,
        scratch_types=[
            pltpu.VMEM((_SCH,), jnp.int32),
            pltpu.VMEM((_SCH, _AF), jnp.float32),
            pltpu.VMEM_SHARED((_N, _AF), jnp.float32),
        ],
    )
    def k(msg_hbm, idx_hbm, z128_hbm, acc_hbm, idx_v, msg_v, acc_sh):
        cid = lax.axis_index("c")
        sid = lax.axis_index("s")
        wid = sid * 2 + cid
        base0 = wid * _PER_W

        @pl.when(sid == 0)
        def _():
            pltpu.sync_copy(z128_hbm, acc_sh)

        plsc.subcore_barrier()

        def body(i, carry):
            base = base0 + i * _SCH
            pltpu.sync_copy(idx_hbm.at[pl.ds(base, _SCH)], idx_v)
            pltpu.sync_copy(msg_hbm.at[pl.ds(base, _SCH)], msg_v)
            pltpu.sync_copy(msg_v, acc_sh.at[idx_v], add=True)
            return carry

        lax.fori_loop(0, _PER_W // _SCH, body, 0)
        plsc.subcore_barrier()

        r0 = pl.multiple_of(sid * 640, 8)

        @pl.when(sid < 15)
        def _():
            pltpu.sync_copy(acc_sh.at[pl.ds(r0, 640)],
                            acc_hbm.at[cid, pl.ds(r0, 640)])

        @pl.when(sid == 15)
        def _():
            pltpu.sync_copy(acc_sh.at[pl.ds(9600, 400)],
                            acc_hbm.at[cid, pl.ds(9600, 400)])

    return k(msg, idx_dst, z128)


def _sc_count(idx_dst, z128):
    mesh = plsc.VectorSubcoreMesh(core_axis_name="c", subcore_axis_name="s")

    @functools.partial(
        pl.kernel,
        out_type=jax.ShapeDtypeStruct((2, _N, _AF), jnp.float32),
        mesh=mesh,
        scratch_types=[
            pltpu.VMEM((_SCH,), jnp.int32),
            pltpu.VMEM((_SCH, _AF), jnp.float32),
            pltpu.VMEM_SHARED((_N, _AF), jnp.float32),
        ],
    )
    def k(idx_hbm, z128_hbm, cnt_hbm, idx_v, ones_v, cnt_sh):
        cid = lax.axis_index("c")
        sid = lax.axis_index("s")
        wid = sid * 2 + cid
        base0 = wid * _PER_W

        @pl.when(sid == 0)
        def _():
            pltpu.sync_copy(z128_hbm, cnt_sh)

        def initones(r, carry):
            ones_v[r, pl.ds(0, 16)] = jnp.ones((16,), jnp.float32)
            ones_v[r, pl.ds(16, 16)] = jnp.ones((16,), jnp.float32)
            ones_v[r, pl.ds(32, 16)] = jnp.ones((16,), jnp.float32)
            ones_v[r, pl.ds(48, 16)] = jnp.ones((16,), jnp.float32)
            ones_v[r, pl.ds(64, 16)] = jnp.ones((16,), jnp.float32)
            ones_v[r, pl.ds(80, 16)] = jnp.ones((16,), jnp.float32)
            ones_v[r, pl.ds(96, 16)] = jnp.ones((16,), jnp.float32)
            ones_v[r, pl.ds(112, 16)] = jnp.ones((16,), jnp.float32)
            return carry

        lax.fori_loop(0, _SCH, initones, 0)
        plsc.subcore_barrier()

        def body(i, carry):
            base = base0 + i * _SCH
            pltpu.sync_copy(idx_hbm.at[pl.ds(base, _SCH)], idx_v)
            pltpu.sync_copy(ones_v, cnt_sh.at[idx_v], add=True)
            return carry

        lax.fori_loop(0, _PER_W // _SCH, body, 0)
        plsc.subcore_barrier()

        r0 = pl.multiple_of(sid * 640, 8)

        @pl.when(sid < 15)
        def _():
            pltpu.sync_copy(cnt_sh.at[pl.ds(r0, 640)],
                            cnt_hbm.at[cid, pl.ds(r0, 640)])

        @pl.when(sid == 15)
        def _():
            pltpu.sync_copy(cnt_sh.at[pl.ds(9600, 400)],
                            cnt_hbm.at[cid, pl.ds(9600, 400)])

    return k(idx_dst, z128)


_SW = 24


def _p1_body(a1_ref, a2_ref, edge_ref, rbf_ref, ce_ref, wrbf_ref, wfull_ref,
             tg_ref, stats_ref):
    @pl.when(pl.program_id(0) == 0)
    def _():
        stats_ref[...] = jnp.zeros_like(stats_ref)

    wa = wfull_ref[0:_AF, :].astype(jnp.bfloat16)
    wb = wfull_ref[_AF:2 * _AF, :].astype(jnp.bfloat16)
    wc = wfull_ref[2 * _AF:, :]
    nbr = edge_ref[...] * jnp.dot(rbf_ref[...], wrbf_ref[...],
                                  preferred_element_type=jnp.float32)
    tg = (jnp.dot(a1_ref[...].astype(jnp.bfloat16), wa,
                  preferred_element_type=jnp.float32)
          + jnp.dot(a2_ref[...].astype(jnp.bfloat16), wb,
                    preferred_element_type=jnp.float32)
          + jnp.dot(nbr, wc, preferred_element_type=jnp.float32))
    tg_ref[...] = tg.astype(jnp.bfloat16)

    ce = ce_ref[...]
    c_lo = jnp.min(ce)
    c_hi = jnp.max(ce)
    base = pl.multiple_of(jnp.minimum(c_lo - (c_lo % 8), _NC - _SW), 8)
    fastp = (c_hi - base) < _SW

    @pl.when(fastp)
    def _():
        iot = lax.broadcasted_iota(jnp.int32, (1, _SW), 1)
        oh = (ce == base + iot).astype(jnp.float32)
        tgcat = jnp.concatenate(
            [tg, tg * tg, jnp.ones((_BE, _AF), jnp.float32)], axis=1)
        upd = lax.dot_general(oh, tgcat, (((0,), (0,)), ((), ())),
                              preferred_element_type=jnp.float32)
        stats_ref[pl.ds(base, _SW), :] += upd

    @pl.when(jnp.logical_not(fastp))
    def _():
        def crystal_iter(c, carry):
            m = (ce == c).astype(jnp.float32)
            mt = m * tg
            s_row = jnp.sum(mt, axis=0, keepdims=True)
            q_row = jnp.sum(mt * tg, axis=0, keepdims=True)
            n_row = jnp.full((1, _AF), jnp.sum(m), jnp.float32)
            upd = jnp.concatenate([s_row, q_row, n_row], axis=1)
            stats_ref[pl.ds(c, 1), :] += upd
            return carry

        lax.fori_loop(c_lo, c_hi + 1, crystal_iter, 0)


def _edge_pass1(a1, a2, edge, rbf, ce2, W_rbf, W_full):
    return pl.pallas_call(
        _p1_body,
        grid=(_NB,),
        in_specs=[
            pl.BlockSpec((_BE, _AF), lambda i: (i, 0)),
            pl.BlockSpec((_BE, _AF), lambda i: (i, 0)),
            pl.BlockSpec((_BE, _NF), lambda i: (i, 0)),
            pl.BlockSpec((_BE, _NR), lambda i: (i, 0)),
            pl.BlockSpec((_BE, 1), lambda i: (i, 0)),
            pl.BlockSpec((_NR, _NF), lambda i: (0, 0)),
            pl.BlockSpec((2 * _AF + _NF, 2 * _AF), lambda i: (0, 0)),
        ],
        out_specs=[
            pl.BlockSpec((_BE, 2 * _AF), lambda i: (i, 0)),
            pl.BlockSpec((_NC, 5 * _AF), lambda i: (0, 0)),
        ],
        out_shape=[
            jax.ShapeDtypeStruct((_E, 2 * _AF), jnp.bfloat16),
            jax.ShapeDtypeStruct((_NC, 5 * _AF), jnp.float32),
        ],
        compiler_params=pltpu.CompilerParams(
            dimension_semantics=("arbitrary",)),
    )(a1, a2, edge, rbf, ce2, W_rbf, W_full)


def _p2_body(tg_ref, ce_ref, stats_ref, g1_ref, b1_ref, wm_ref, msg_ref):
    tg = tg_ref[...].astype(jnp.float32)
    ce = ce_ref[...]
    c_lo = jnp.min(ce)
    c_hi = jnp.max(ce)
    gamma = g1_ref[...]
    beta = b1_ref[...]
    base = pl.multiple_of(jnp.minimum(c_lo - (c_lo % 8), _NC - _SW), 8)
    fastp = (c_hi - base) < _SW

    def affine_rows(rows):
        n = jnp.maximum(rows[:, 4 * _AF:4 * _AF + 1], 1.0)
        srow = rows[:, 0:2 * _AF] / n
        qrow = rows[:, 2 * _AF:4 * _AF] / n
        var = jnp.maximum(qrow - srow * srow, 0.0)
        a = gamma * lax.rsqrt(var + _EPS)
        b = beta - srow * a
        return a, b

    def fast():
        rows = stats_ref[pl.ds(base, _SW), :]
        a, b = affine_rows(rows)
        iot = lax.broadcasted_iota(jnp.int32, (1, _SW), 1)
        oh = (ce == base + iot).astype(jnp.float32)
        ae = jnp.dot(oh, a, preferred_element_type=jnp.float32)
        be = jnp.dot(oh, b, preferred_element_type=jnp.float32)
        return ae, be

    def slow():
        def crystal_iter(c, carry):
            ae, be = carry
            a, b = affine_rows(stats_ref[pl.ds(c, 1), :])
            m = (ce == c).astype(jnp.float32)
            return ae + m * a, be + m * b

        z = jnp.zeros((_BE, 2 * _AF), jnp.float32)
        return lax.fori_loop(c_lo, c_hi + 1, crystal_iter, (z, z))

    ae, be = lax.cond(fastp, fast, slow)
    tgn = tg * ae + be
    filt = jax.nn.sigmoid(jnp.dot(tgn[:, :_AF], wm_ref[...],
                                  preferred_element_type=jnp.float32))
    core = jnp.maximum(tgn[:, _AF:], 0.0)
    msg_ref[...] = filt * core


def _edge_pass2(tg, ce2, stats, gamma1, beta1, W_mask):
    return pl.pallas_call(
        _p2_body,
        grid=(_NB,),
        in_specs=[
            pl.BlockSpec((_BE, 2 * _AF), lambda i: (i, 0)),
            pl.BlockSpec((_BE, 1), lambda i: (i, 0)),
            pl.BlockSpec((_NC, 5 * _AF), lambda i: (0, 0)),
            pl.BlockSpec((1, 2 * _AF), lambda i: (0, 0)),
            pl.BlockSpec((1, 2 * _AF), lambda i: (0, 0)),
            pl.BlockSpec((_AF, 1), lambda i: (0, 0)),
        ],
        out_specs=pl.BlockSpec((_BE, _AF), lambda i: (i, 0)),
        out_shape=jax.ShapeDtypeStruct((_E, _AF), jnp.float32),
        compiler_params=pltpu.CompilerParams(
            dimension_semantics=("arbitrary",)),
    )(tg, ce2, stats, gamma1, beta1, W_mask)


def _fin_body(acc_ref, cnt_ref, af_ref, ca_ref, g2_ref, b2_ref,
              w10_ref, bb10_ref, w20_ref, bb20_ref,
              w11_ref, bb11_ref, w21_ref, bb21_ref, out_ref):
    acc = acc_ref[0] + acc_ref[1]
    cnt = cnt_ref[0, :, 0:1] + cnt_ref[1, :, 0:1]
    x = acc / jnp.maximum(cnt, 1.0)

    ca = ca_ref[...]
    iot = lax.broadcasted_iota(jnp.int32, (1, _NC), 1)
    onehot = (ca == iot).astype(jnp.float32)
    xcat = jnp.concatenate([x, x * x], axis=1)
    sums = lax.dot_general(onehot, xcat, (((0,), (0,)), ((), ())),
                           preferred_element_type=jnp.float32)
    cntc = jnp.maximum(
        jnp.sum(onehot, axis=0, keepdims=True), 1.0).reshape(_NC, 1)
    mean = sums[:, :_AF] / cntc
    var = jnp.maximum(sums[:, _AF:] / cntc - mean * mean, 0.0)
    ac = g2_ref[...] * lax.rsqrt(var + _EPS)
    bc = b2_ref[...] - mean * ac
    a_e = jnp.dot(onehot, ac, preferred_element_type=jnp.float32)
    b_e = jnp.dot(onehot, bc, preferred_element_type=jnp.float32)
    y = x * a_e + b_e

    h = jnp.maximum(jnp.dot(y, w10_ref[...],
                            preferred_element_type=jnp.float32)
                    + bb10_ref[...], 0.0)
    y = y + jnp.dot(h, w20_ref[...],
                    preferred_element_type=jnp.float32) + bb20_ref[...]
    h = jnp.maximum(jnp.dot(y, w11_ref[...],
                            preferred_element_type=jnp.float32)
                    + bb11_ref[...], 0.0)
    y = y + jnp.dot(h, w21_ref[...],
                    preferred_element_type=jnp.float32) + bb21_ref[...]

    out_ref[...] = 0.7071067811865476 * jnp.maximum(af_ref[...] + y, 0.0)


def _final(acc, cnt, atom_fea, ca2, gamma2, beta2,
           w10, b10, w20, b20, w11, b11, w21, b21):
    full = lambda s: pl.BlockSpec(s, lambda: tuple(0 for _ in s))
    return pl.pallas_call(
        _fin_body,
        in_specs=[
            full((2, _N, _AF)), full((2, _N, _AF)), full((_N, _AF)),
            full((_N, 1)), full((1, _AF)), full((1, _AF)),
            full((_AF, _AF // 2)), full((1, _AF // 2)),
            full((_AF // 2, _AF)), full((1, _AF)),
            full((_AF, _AF // 2)), full((1, _AF // 2)),
            full((_AF // 2, _AF)), full((1, _AF)),
        ],
        out_specs=full((_N, _AF)),
        out_shape=jax.ShapeDtypeStruct((_N, _AF), jnp.float32),
    )(acc, cnt, atom_fea, ca2, gamma2, beta2,
      w10, b10, w20, b20, w11, b11, w21, b21)


def kernel(atom_fea, edge, rbf, nbr_fea_idx, crystal_atom_idx,
           crystal_edge_idx, W_rbf, W_full, W_mask, gamma1, beta1, gamma2,
           beta2, res_W1_0, res_b1_0, res_W2_0, res_b2_0, res_W1_1,
           res_b1_1, res_W2_1, res_b2_1):
    idx_dst = nbr_fea_idx[:, 0]
    idx_src = nbr_fea_idx[:, 1]

    a1, a2 = _sc_gather(atom_fea, idx_dst, idx_src)

    ce2 = crystal_edge_idx.reshape(_E, 1)
    tg, stats = _edge_pass1(a1, a2, edge, rbf, ce2, W_rbf, W_full)
    msg = _edge_pass2(tg, ce2, stats, gamma1.reshape(1, -1),
                      beta1.reshape(1, -1), W_mask)

    z128 = jnp.zeros((_N, _AF), jnp.float32)
    acc = _sc_scatter(msg, idx_dst, z128)
    cnt_at = _sc_count(idx_dst, z128)

    out = _final(acc, cnt_at, atom_fea, crystal_atom_idx.reshape(_N, 1),
                 gamma2.reshape(1, -1), beta2.reshape(1, -1),
                 res_W1_0, res_b1_0.reshape(1, -1),
                 res_W2_0, res_b2_0.reshape(1, -1),
                 res_W1_1, res_b1_1.reshape(1, -1),
                 res_W2_1, res_b2_1.reshape(1, -1))
    return out


# speedup vs baseline: 5.8157x; 1.0271x over previous
import functools

import jax
import jax.numpy as jnp
from jax import lax
from jax.experimental import pallas as pl
from jax.experimental.pallas import tpu as pltpu
from jax.experimental.pallas import tpu_sc as plsc

_N = 10000
_E = 320000
_AF = 128
_NF = 16
_NR = 16
_NC = 256
_EPS = 1e-5

_NW = 32
_PER_W = _E // _NW
_GCH = 80
_SCH = 80
_ROWS_T = _N // 16

_BE = 2000
_NB = _E // _BE


_GC2 = 128
_NGF = _PER_W // _GC2
_GTL = _PER_W - _NGF * _GC2


def _sc_gather(atom_fea, idx_dst, idx_src):
    mesh = plsc.VectorSubcoreMesh(core_axis_name="c", subcore_axis_name="s")

    @functools.partial(
        pl.kernel,
        out_type=(
            jax.ShapeDtypeStruct((_E, _AF), jnp.float32),
            jax.ShapeDtypeStruct((_E, _AF), jnp.float32),
        ),
        mesh=mesh,
        scratch_types=[
            pltpu.VMEM((_GC2,), jnp.int32), pltpu.VMEM((_GC2,), jnp.int32),
            pltpu.VMEM((_GC2,), jnp.int32), pltpu.VMEM((_GC2,), jnp.int32),
            pltpu.VMEM((_GC2, _AF), jnp.float32),
            pltpu.VMEM((_GC2, _AF), jnp.float32),
            pltpu.VMEM((_GC2, _AF), jnp.float32),
            pltpu.VMEM((_GC2, _AF), jnp.float32),
            pltpu.VMEM((_GTL,), jnp.int32), pltpu.VMEM((_GTL,), jnp.int32),
            pltpu.VMEM((_GTL, _AF), jnp.float32),
            pltpu.VMEM((_GTL, _AF), jnp.float32),
            pltpu.SemaphoreType.DMA, pltpu.SemaphoreType.DMA,
            pltpu.SemaphoreType.DMA, pltpu.SemaphoreType.DMA,
            pltpu.SemaphoreType.DMA, pltpu.SemaphoreType.DMA,
            pltpu.SemaphoreType.DMA, pltpu.SemaphoreType.DMA,
        ],
    )
    def k(atom_hbm, i1_hbm, i2_hbm, o1_hbm, o2_hbm,
          i1a, i2a, i1b, i2b, b1a, b2a, b1b, b2b, it1, it2, tb1, tb2,
          s1a, s2a, s1b, s2b, w1a, w2a, w1b, w2b):
        wid = lax.axis_index("s") * 2 + lax.axis_index("c")
        base0 = wid * _PER_W
        bufs = ((i1a, i2a, b1a, b2a, s1a, s2a, w1a, w2a),
                (i1b, i2b, b1b, b2b, s1b, s2b, w1b, w2b))

        def run_chunk(p, base, first):
            i1v, i2v, b1, b2, s1, s2, w1, w2 = bufs[p]

            @pl.when(jnp.logical_not(first))
            def _():
                pltpu.make_async_copy(b1, o1_hbm.at[pl.ds(base0, _GC2)],
                                      w1).wait()
                pltpu.make_async_copy(b2, o2_hbm.at[pl.ds(base0, _GC2)],
                                      w2).wait()

            pltpu.sync_copy(i1_hbm.at[pl.ds(base, _GC2)], i1v)
            pltpu.sync_copy(i2_hbm.at[pl.ds(base, _GC2)], i2v)
            c1 = pltpu.async_copy(atom_hbm.at[i1v], b1, s1)
            c2 = pltpu.async_copy(atom_hbm.at[i2v], b2, s2)
            c1.wait()
            c2.wait()
            pltpu.async_copy(b1, o1_hbm.at[pl.ds(base, _GC2)], w1)
            pltpu.async_copy(b2, o2_hbm.at[pl.ds(base, _GC2)], w2)

        def body2(t, carry):
            run_chunk(0, base0 + (2 * t) * _GC2, t == 0)
            run_chunk(1, base0 + (2 * t + 1) * _GC2, t == 0)
            return carry

        lax.fori_loop(0, _NGF // 2, body2, 0)
        for p in (0, 1):
            _, _, b1, b2, _, _, w1, w2 = bufs[p]
            pltpu.make_async_copy(b1, o1_hbm.at[pl.ds(base0, _GC2)], w1).wait()
            pltpu.make_async_copy(b2, o2_hbm.at[pl.ds(base0, _GC2)], w2).wait()

        tbase = base0 + _NGF * _GC2
        pltpu.sync_copy(i1_hbm.at[pl.ds(tbase, _GTL)], it1)
        pltpu.sync_copy(i2_hbm.at[pl.ds(tbase, _GTL)], it2)
        c1 = pltpu.async_copy(atom_hbm.at[it1], tb1, s1a)
        c2 = pltpu.async_copy(atom_hbm.at[it2], tb2, s2a)
        c1.wait()
        c2.wait()
        pltpu.sync_copy(tb1, o1_hbm.at[pl.ds(tbase, _GTL)])
        pltpu.sync_copy(tb2, o2_hbm.at[pl.ds(tbase, _GTL)])

    return k(atom_fea, idx_dst, idx_src)


def _sc_scatter(msg, idx_dst, z128):
    mesh = plsc.VectorSubcoreMesh(core_axis_name="c", subcore_axis_name="s")

    @functools.partial(
        pl.kernel,
        out_type=jax.ShapeDtypeStruct((2, _N, _AF), jnp.float32),
        mesh=mesh,
        scratch_types=[
            pltpu.VMEM((_SCH,), jnp.int32),
            pltpu.VMEM((_SCH, _AF), jnp.float32),
            pltpu.VMEM_SHARED((_N, _AF), jnp.float32),
        ],
    )
    def k(msg_hbm, idx_hbm, z128_hbm, acc_hbm, idx_v, msg_v, acc_sh):
        cid = lax.axis_index("c")
        sid = lax.axis_index("s")
        wid = sid * 2 + cid
        base0 = wid * _PER_W

        @pl.when(sid == 0)
        def _():
            pltpu.sync_copy(z128_hbm, acc_sh)

        plsc.subcore_barrier()

        def body(i, carry):
            base = base0 + i * _SCH
            pltpu.sync_copy(idx_hbm.at[pl.ds(base, _SCH)], idx_v)
            pltpu.sync_copy(msg_hbm.at[pl.ds(base, _SCH)], msg_v)
            pltpu.sync_copy(msg_v, acc_sh.at[idx_v], add=True)
            return carry

        lax.fori_loop(0, _PER_W // _SCH, body, 0)
        plsc.subcore_barrier()

        r0 = pl.multiple_of(sid * 640, 8)

        @pl.when(sid < 15)
        def _():
            pltpu.sync_copy(acc_sh.at[pl.ds(r0, 640)],
                            acc_hbm.at[cid, pl.ds(r0, 640)])

        @pl.when(sid == 15)
        def _():
            pltpu.sync_copy(acc_sh.at[pl.ds(9600, 400)],
                            acc_hbm.at[cid, pl.ds(9600, 400)])

    return k(msg, idx_dst, z128)


def _sc_count(idx_dst, z128):
    mesh = plsc.VectorSubcoreMesh(core_axis_name="c", subcore_axis_name="s")

    @functools.partial(
        pl.kernel,
        out_type=jax.ShapeDtypeStruct((2, _N, _AF), jnp.float32),
        mesh=mesh,
        scratch_types=[
            pltpu.VMEM((_SCH,), jnp.int32),
            pltpu.VMEM((_SCH, _AF), jnp.float32),
            pltpu.VMEM_SHARED((_N, _AF), jnp.float32),
        ],
    )
    def k(idx_hbm, z128_hbm, cnt_hbm, idx_v, ones_v, cnt_sh):
        cid = lax.axis_index("c")
        sid = lax.axis_index("s")
        wid = sid * 2 + cid
        base0 = wid * _PER_W

        @pl.when(sid == 0)
        def _():
            pltpu.sync_copy(z128_hbm, cnt_sh)

        def initones(r, carry):
            ones_v[r, pl.ds(0, 16)] = jnp.ones((16,), jnp.float32)
            ones_v[r, pl.ds(16, 16)] = jnp.ones((16,), jnp.float32)
            ones_v[r, pl.ds(32, 16)] = jnp.ones((16,), jnp.float32)
            ones_v[r, pl.ds(48, 16)] = jnp.ones((16,), jnp.float32)
            ones_v[r, pl.ds(64, 16)] = jnp.ones((16,), jnp.float32)
            ones_v[r, pl.ds(80, 16)] = jnp.ones((16,), jnp.float32)
            ones_v[r, pl.ds(96, 16)] = jnp.ones((16,), jnp.float32)
            ones_v[r, pl.ds(112, 16)] = jnp.ones((16,), jnp.float32)
            return carry

        lax.fori_loop(0, _SCH, initones, 0)
        plsc.subcore_barrier()

        def body(i, carry):
            base = base0 + i * _SCH
            pltpu.sync_copy(idx_hbm.at[pl.ds(base, _SCH)], idx_v)
            pltpu.sync_copy(ones_v, cnt_sh.at[idx_v], add=True)
            return carry

        lax.fori_loop(0, _PER_W // _SCH, body, 0)
        plsc.subcore_barrier()

        r0 = pl.multiple_of(sid * 640, 8)

        @pl.when(sid < 15)
        def _():
            pltpu.sync_copy(cnt_sh.at[pl.ds(r0, 640)],
                            cnt_hbm.at[cid, pl.ds(r0, 640)])

        @pl.when(sid == 15)
        def _():
            pltpu.sync_copy(cnt_sh.at[pl.ds(9600, 400)],
                            cnt_hbm.at[cid, pl.ds(9600, 400)])

    return k(idx_dst, z128)


_SW = 24


def _p1_body(a1_ref, a2_ref, edge_ref, rbf_ref, ce_ref, wrbf_ref, wfull_ref,
             tg_ref, stats_ref):
    @pl.when(pl.program_id(0) == 0)
    def _():
        stats_ref[...] = jnp.zeros_like(stats_ref)

    wa = wfull_ref[0:_AF, :].astype(jnp.bfloat16)
    wb = wfull_ref[_AF:2 * _AF, :].astype(jnp.bfloat16)
    wc = wfull_ref[2 * _AF:, :]
    nbr = edge_ref[...] * jnp.dot(rbf_ref[...], wrbf_ref[...],
                                  preferred_element_type=jnp.float32)
    tg = (jnp.dot(a1_ref[...].astype(jnp.bfloat16), wa,
                  preferred_element_type=jnp.float32)
          + jnp.dot(a2_ref[...].astype(jnp.bfloat16), wb,
                    preferred_element_type=jnp.float32)
          + jnp.dot(nbr, wc, preferred_element_type=jnp.float32))
    tg_ref[...] = tg.astype(jnp.bfloat16)

    ce = ce_ref[...]
    c_lo = jnp.min(ce)
    c_hi = jnp.max(ce)
    base = pl.multiple_of(jnp.minimum(c_lo - (c_lo % 8), _NC - _SW), 8)
    fastp = (c_hi - base) < _SW

    @pl.when(fastp)
    def _():
        iot = lax.broadcasted_iota(jnp.int32, (1, _SW), 1)
        oh = (ce == base + iot).astype(jnp.float32)
        tgcat = jnp.concatenate(
            [tg, tg * tg, jnp.ones((_BE, _AF), jnp.float32)], axis=1)
        upd = lax.dot_general(oh, tgcat, (((0,), (0,)), ((), ())),
                              preferred_element_type=jnp.float32)
        stats_ref[pl.ds(base, _SW), :] += upd

    @pl.when(jnp.logical_not(fastp))
    def _():
        def crystal_iter(c, carry):
            m = (ce == c).astype(jnp.float32)
            mt = m * tg
            s_row = jnp.sum(mt, axis=0, keepdims=True)
            q_row = jnp.sum(mt * tg, axis=0, keepdims=True)
            n_row = jnp.full((1, _AF), jnp.sum(m), jnp.float32)
            upd = jnp.concatenate([s_row, q_row, n_row], axis=1)
            stats_ref[pl.ds(c, 1), :] += upd
            return carry

        lax.fori_loop(c_lo, c_hi + 1, crystal_iter, 0)


def _edge_pass1(a1, a2, edge, rbf, ce2, W_rbf, W_full):
    return pl.pallas_call(
        _p1_body,
        grid=(_NB,),
        in_specs=[
            pl.BlockSpec((_BE, _AF), lambda i: (i, 0)),
            pl.BlockSpec((_BE, _AF), lambda i: (i, 0)),
            pl.BlockSpec((_BE, _NF), lambda i: (i, 0)),
            pl.BlockSpec((_BE, _NR), lambda i: (i, 0)),
            pl.BlockSpec((_BE, 1), lambda i: (i, 0)),
            pl.BlockSpec((_NR, _NF), lambda i: (0, 0)),
            pl.BlockSpec((2 * _AF + _NF, 2 * _AF), lambda i: (0, 0)),
        ],
        out_specs=[
            pl.BlockSpec((_BE, 2 * _AF), lambda i: (i, 0)),
            pl.BlockSpec((_NC, 5 * _AF), lambda i: (0, 0)),
        ],
        out_shape=[
            jax.ShapeDtypeStruct((_E, 2 * _AF), jnp.bfloat16),
            jax.ShapeDtypeStruct((_NC, 5 * _AF), jnp.float32),
        ],
        compiler_params=pltpu.CompilerParams(
            dimension_semantics=("arbitrary",)),
    )(a1, a2, edge, rbf, ce2, W_rbf, W_full)


def _p2_body(tg_ref, ce_ref, stats_ref, g1_ref, b1_ref, wm_ref, msg_ref):
    tg = tg_ref[...].astype(jnp.float32)
    ce = ce_ref[...]
    c_lo = jnp.min(ce)
    c_hi = jnp.max(ce)
    gamma = g1_ref[...]
    beta = b1_ref[...]
    base = pl.multiple_of(jnp.minimum(c_lo - (c_lo % 8), _NC - _SW), 8)
    fastp = (c_hi - base) < _SW

    def affine_rows(rows):
        n = jnp.maximum(rows[:, 4 * _AF:4 * _AF + 1], 1.0)
        srow = rows[:, 0:2 * _AF] / n
        qrow = rows[:, 2 * _AF:4 * _AF] / n
        var = jnp.maximum(qrow - srow * srow, 0.0)
        a = gamma * lax.rsqrt(var + _EPS)
        b = beta - srow * a
        return a, b

    def fast():
        rows = stats_ref[pl.ds(base, _SW), :]
        a, b = affine_rows(rows)
        iot = lax.broadcasted_iota(jnp.int32, (1, _SW), 1)
        oh = (ce == base + iot).astype(jnp.float32)
        ae = jnp.dot(oh, a, preferred_element_type=jnp.float32)
        be = jnp.dot(oh, b, preferred_element_type=jnp.float32)
        return ae, be

    def slow():
        def crystal_iter(c, carry):
            ae, be = carry
            a, b = affine_rows(stats_ref[pl.ds(c, 1), :])
            m = (ce == c).astype(jnp.float32)
            return ae + m * a, be + m * b

        z = jnp.zeros((_BE, 2 * _AF), jnp.float32)
        return lax.fori_loop(c_lo, c_hi + 1, crystal_iter, (z, z))

    ae, be = lax.cond(fastp, fast, slow)
    tgn = tg * ae + be
    filt = jax.nn.sigmoid(jnp.dot(tgn[:, :_AF], wm_ref[...],
                                  preferred_element_type=jnp.float32))
    core = jnp.maximum(tgn[:, _AF:], 0.0)
    msg_ref[...] = filt * core


def _edge_pass2(tg, ce2, stats, gamma1, beta1, W_mask):
    return pl.pallas_call(
        _p2_body,
        grid=(_NB,),
        in_specs=[
            pl.BlockSpec((_BE, 2 * _AF), lambda i: (i, 0)),
            pl.BlockSpec((_BE, 1), lambda i: (i, 0)),
            pl.BlockSpec((_NC, 5 * _AF), lambda i: (0, 0)),
            pl.BlockSpec((1, 2 * _AF), lambda i: (0, 0)),
            pl.BlockSpec((1, 2 * _AF), lambda i: (0, 0)),
            pl.BlockSpec((_AF, 1), lambda i: (0, 0)),
        ],
        out_specs=pl.BlockSpec((_BE, _AF), lambda i: (i, 0)),
        out_shape=jax.ShapeDtypeStruct((_E, _AF), jnp.float32),
        compiler_params=pltpu.CompilerParams(
            dimension_semantics=("arbitrary",)),
    )(tg, ce2, stats, gamma1, beta1, W_mask)


def _fin_body(acc_ref, cnt_ref, af_ref, ca_ref, g2_ref, b2_ref,
              w10_ref, bb10_ref, w20_ref, bb20_ref,
              w11_ref, bb11_ref, w21_ref, bb21_ref, out_ref):
    acc = acc_ref[0] + acc_ref[1]
    cnt = cnt_ref[0, :, 0:1] + cnt_ref[1, :, 0:1]
    x = acc / jnp.maximum(cnt, 1.0)

    ca = ca_ref[...]
    iot = lax.broadcasted_iota(jnp.int32, (1, _NC), 1)
    onehot = (ca == iot).astype(jnp.float32)
    xcat = jnp.concatenate([x, x * x], axis=1)
    sums = lax.dot_general(onehot, xcat, (((0,), (0,)), ((), ())),
                           preferred_element_type=jnp.float32)
    cntc = jnp.maximum(
        jnp.sum(onehot, axis=0, keepdims=True), 1.0).reshape(_NC, 1)
    mean = sums[:, :_AF] / cntc
    var = jnp.maximum(sums[:, _AF:] / cntc - mean * mean, 0.0)
    ac = g2_ref[...] * lax.rsqrt(var + _EPS)
    bc = b2_ref[...] - mean * ac
    a_e = jnp.dot(onehot, ac, preferred_element_type=jnp.float32)
    b_e = jnp.dot(onehot, bc, preferred_element_type=jnp.float32)
    y = x * a_e + b_e

    h = jnp.maximum(jnp.dot(y, w10_ref[...],
                            preferred_element_type=jnp.float32)
                    + bb10_ref[...], 0.0)
    y = y + jnp.dot(h, w20_ref[...],
                    preferred_element_type=jnp.float32) + bb20_ref[...]
    h = jnp.maximum(jnp.dot(y, w11_ref[...],
                            preferred_element_type=jnp.float32)
                    + bb11_ref[...], 0.0)
    y = y + jnp.dot(h, w21_ref[...],
                    preferred_element_type=jnp.float32) + bb21_ref[...]

    out_ref[...] = 0.7071067811865476 * jnp.maximum(af_ref[...] + y, 0.0)


def _final(acc, cnt, atom_fea, ca2, gamma2, beta2,
           w10, b10, w20, b20, w11, b11, w21, b21):
    full = lambda s: pl.BlockSpec(s, lambda: tuple(0 for _ in s))
    return pl.pallas_call(
        _fin_body,
        in_specs=[
            full((2, _N, _AF)), full((2, _N, _AF)), full((_N, _AF)),
            full((_N, 1)), full((1, _AF)), full((1, _AF)),
            full((_AF, _AF // 2)), full((1, _AF // 2)),
            full((_AF // 2, _AF)), full((1, _AF)),
            full((_AF, _AF // 2)), full((1, _AF // 2)),
            full((_AF // 2, _AF)), full((1, _AF)),
        ],
        out_specs=full((_N, _AF)),
        out_shape=jax.ShapeDtypeStruct((_N, _AF), jnp.float32),
    )(acc, cnt, atom_fea, ca2, gamma2, beta2,
      w10, b10, w20, b20, w11, b11, w21, b21)


def kernel(atom_fea, edge, rbf, nbr_fea_idx, crystal_atom_idx,
           crystal_edge_idx, W_rbf, W_full, W_mask, gamma1, beta1, gamma2,
           beta2, res_W1_0, res_b1_0, res_W2_0, res_b2_0, res_W1_1,
           res_b1_1, res_W2_1, res_b2_1):
    idx_dst = nbr_fea_idx[:, 0]
    idx_src = nbr_fea_idx[:, 1]

    a1, a2 = _sc_gather(atom_fea, idx_dst, idx_src)

    ce2 = crystal_edge_idx.reshape(_E, 1)
    tg, stats = _edge_pass1(a1, a2, edge, rbf, ce2, W_rbf, W_full)
    msg = _edge_pass2(tg, ce2, stats, gamma1.reshape(1, -1),
                      beta1.reshape(1, -1), W_mask)

    z128 = jnp.zeros((_N, _AF), jnp.float32)
    acc = _sc_scatter(msg, idx_dst, z128)
    cnt_at = _sc_count(idx_dst, z128)

    out = _final(acc, cnt_at, atom_fea, crystal_atom_idx.reshape(_N, 1),
                 gamma2.reshape(1, -1), beta2.reshape(1, -1),
                 res_W1_0, res_b1_0.reshape(1, -1),
                 res_W2_0, res_b2_0.reshape(1, -1),
                 res_W1_1, res_b1_1.reshape(1, -1),
                 res_W2_1, res_b2_1.reshape(1, -1))
    return out


# speedup vs baseline: 6.3550x; 1.0927x over previous
import functools

import jax
import jax.numpy as jnp
from jax import lax
from jax.experimental import pallas as pl
from jax.experimental.pallas import tpu as pltpu
from jax.experimental.pallas import tpu_sc as plsc

_N = 10000
_E = 320000
_AF = 128
_NF = 16
_NR = 16
_NC = 256
_EPS = 1e-5

_NW = 32
_PER_W = _E // _NW
_GCH = 80
_SCH = 80
_ROWS_T = _N // 16

_BE = 2000
_NB = _E // _BE


_GC2 = 128
_NGF = _PER_W // _GC2
_GTL = _PER_W - _NGF * _GC2


def _sc_gather(atom_fea, idx_dst, idx_src):
    mesh = plsc.VectorSubcoreMesh(core_axis_name="c", subcore_axis_name="s")

    @functools.partial(
        pl.kernel,
        out_type=(
            jax.ShapeDtypeStruct((_E, _AF), jnp.float32),
            jax.ShapeDtypeStruct((_E, _AF), jnp.float32),
        ),
        mesh=mesh,
        scratch_types=[
            pltpu.VMEM((_GC2,), jnp.int32), pltpu.VMEM((_GC2,), jnp.int32),
            pltpu.VMEM((_GC2,), jnp.int32), pltpu.VMEM((_GC2,), jnp.int32),
            pltpu.VMEM((_GC2, _AF), jnp.float32),
            pltpu.VMEM((_GC2, _AF), jnp.float32),
            pltpu.VMEM((_GC2, _AF), jnp.float32),
            pltpu.VMEM((_GC2, _AF), jnp.float32),
            pltpu.VMEM((_GTL,), jnp.int32), pltpu.VMEM((_GTL,), jnp.int32),
            pltpu.VMEM((_GTL, _AF), jnp.float32),
            pltpu.VMEM((_GTL, _AF), jnp.float32),
            pltpu.SemaphoreType.DMA, pltpu.SemaphoreType.DMA,
            pltpu.SemaphoreType.DMA, pltpu.SemaphoreType.DMA,
            pltpu.SemaphoreType.DMA, pltpu.SemaphoreType.DMA,
            pltpu.SemaphoreType.DMA, pltpu.SemaphoreType.DMA,
        ],
    )
    def k(atom_hbm, i1_hbm, i2_hbm, o1_hbm, o2_hbm,
          i1a, i2a, i1b, i2b, b1a, b2a, b1b, b2b, it1, it2, tb1, tb2,
          s1a, s2a, s1b, s2b, w1a, w2a, w1b, w2b):
        wid = lax.axis_index("s") * 2 + lax.axis_index("c")
        base0 = wid * _PER_W
        bufs = ((i1a, i2a, b1a, b2a, s1a, s2a, w1a, w2a),
                (i1b, i2b, b1b, b2b, s1b, s2b, w1b, w2b))

        def start_gather(p, base):
            i1v, i2v, b1, b2, s1, s2, _, _ = bufs[p]
            pltpu.sync_copy(i1_hbm.at[pl.ds(base, _GC2)], i1v)
            pltpu.sync_copy(i2_hbm.at[pl.ds(base, _GC2)], i2v)
            pltpu.async_copy(atom_hbm.at[i1v], b1, s1)
            pltpu.async_copy(atom_hbm.at[i2v], b2, s2)

        def drain_gather(p, base):
            i1v, i2v, b1, b2, s1, s2, w1, w2 = bufs[p]
            pltpu.make_async_copy(atom_hbm.at[i1v], b1, s1).wait()
            pltpu.make_async_copy(atom_hbm.at[i2v], b2, s2).wait()
            pltpu.async_copy(b1, o1_hbm.at[pl.ds(base, _GC2)], w1)
            pltpu.async_copy(b2, o2_hbm.at[pl.ds(base, _GC2)], w2)

        def wait_wb(p):
            _, _, b1, b2, _, _, w1, w2 = bufs[p]
            pltpu.make_async_copy(b1, o1_hbm.at[pl.ds(base0, _GC2)], w1).wait()
            pltpu.make_async_copy(b2, o2_hbm.at[pl.ds(base0, _GC2)], w2).wait()

        start_gather(0, base0)

        def step(p, j):
            @pl.when(j >= 2)
            def _():
                wait_wb(p)

            start_gather(p, base0 + j * _GC2)
            drain_gather(1 - p, base0 + (j - 1) * _GC2)

        def body(j, carry):
            @pl.when(j % 2 == 1)
            def _():
                step(1, j)

            @pl.when(j % 2 == 0)
            def _():
                step(0, j)

            return carry

        lax.fori_loop(1, _NGF, body, 0)
        drain_gather((_NGF - 1) % 2, base0 + (_NGF - 1) * _GC2)
        wait_wb(0)
        wait_wb(1)

        tbase = base0 + _NGF * _GC2
        pltpu.sync_copy(i1_hbm.at[pl.ds(tbase, _GTL)], it1)
        pltpu.sync_copy(i2_hbm.at[pl.ds(tbase, _GTL)], it2)
        c1 = pltpu.async_copy(atom_hbm.at[it1], tb1, s1a)
        c2 = pltpu.async_copy(atom_hbm.at[it2], tb2, s2a)
        c1.wait()
        c2.wait()
        pltpu.sync_copy(tb1, o1_hbm.at[pl.ds(tbase, _GTL)])
        pltpu.sync_copy(tb2, o2_hbm.at[pl.ds(tbase, _GTL)])

    return k(atom_fea, idx_dst, idx_src)


def _sc_scatter(msg, idx_dst, z128):
    mesh = plsc.VectorSubcoreMesh(core_axis_name="c", subcore_axis_name="s")

    @functools.partial(
        pl.kernel,
        out_type=jax.ShapeDtypeStruct((2, _N, _AF), jnp.float32),
        mesh=mesh,
        scratch_types=[
            pltpu.VMEM((_SCH,), jnp.int32),
            pltpu.VMEM((_SCH,), jnp.int32),
            pltpu.VMEM((_SCH, _AF), jnp.float32),
            pltpu.VMEM((_SCH, _AF), jnp.float32),
            pltpu.VMEM_SHARED((_N, _AF), jnp.float32),
            pltpu.SemaphoreType.DMA, pltpu.SemaphoreType.DMA,
            pltpu.SemaphoreType.DMA, pltpu.SemaphoreType.DMA,
        ],
    )
    def k(msg_hbm, idx_hbm, z128_hbm, acc_hbm, idx_a, idx_b, msg_a, msg_b,
          acc_sh, la, lb, ma, mb):
        cid = lax.axis_index("c")
        sid = lax.axis_index("s")
        wid = sid * 2 + cid
        base0 = wid * _PER_W
        nch = _PER_W // _SCH
        bufs = ((idx_a, msg_a, la, ma), (idx_b, msg_b, lb, mb))

        @pl.when(sid == 0)
        def _():
            pltpu.sync_copy(z128_hbm, acc_sh)

        def start_load(p, base):
            idx_v, msg_v, ls, ms = bufs[p]
            pltpu.async_copy(idx_hbm.at[pl.ds(base, _SCH)], idx_v, ls)
            pltpu.async_copy(msg_hbm.at[pl.ds(base, _SCH)], msg_v, ms)

        def do_scatter(p, base):
            idx_v, msg_v, ls, ms = bufs[p]
            pltpu.make_async_copy(idx_hbm.at[pl.ds(base, _SCH)],
                                  idx_v, ls).wait()
            pltpu.make_async_copy(msg_hbm.at[pl.ds(base, _SCH)],
                                  msg_v, ms).wait()
            pltpu.sync_copy(msg_v, acc_sh.at[idx_v], add=True)

        plsc.subcore_barrier()
        start_load(0, base0)

        def body(j, carry):
            @pl.when(j % 2 == 1)
            def _():
                start_load(1, base0 + j * _SCH)
                do_scatter(0, base0 + (j - 1) * _SCH)

            @pl.when(j % 2 == 0)
            def _():
                start_load(0, base0 + j * _SCH)
                do_scatter(1, base0 + (j - 1) * _SCH)

            return carry

        lax.fori_loop(1, nch, body, 0)
        do_scatter((nch - 1) % 2, base0 + (nch - 1) * _SCH)
        plsc.subcore_barrier()

        r0 = pl.multiple_of(sid * 640, 8)

        @pl.when(sid < 15)
        def _():
            pltpu.sync_copy(acc_sh.at[pl.ds(r0, 640)],
                            acc_hbm.at[cid, pl.ds(r0, 640)])

        @pl.when(sid == 15)
        def _():
            pltpu.sync_copy(acc_sh.at[pl.ds(9600, 400)],
                            acc_hbm.at[cid, pl.ds(9600, 400)])

    return k(msg, idx_dst, z128)


def _sc_count(idx_dst, z128):
    mesh = plsc.VectorSubcoreMesh(core_axis_name="c", subcore_axis_name="s")

    @functools.partial(
        pl.kernel,
        out_type=jax.ShapeDtypeStruct((2, _N, _AF), jnp.float32),
        mesh=mesh,
        scratch_types=[
            pltpu.VMEM((_SCH,), jnp.int32),
            pltpu.VMEM((_SCH, _AF), jnp.float32),
            pltpu.VMEM_SHARED((_N, _AF), jnp.float32),
        ],
    )
    def k(idx_hbm, z128_hbm, cnt_hbm, idx_v, ones_v, cnt_sh):
        cid = lax.axis_index("c")
        sid = lax.axis_index("s")
        wid = sid * 2 + cid
        base0 = wid * _PER_W

        @pl.when(sid == 0)
        def _():
            pltpu.sync_copy(z128_hbm, cnt_sh)

        def initones(r, carry):
            ones_v[r, pl.ds(0, 16)] = jnp.ones((16,), jnp.float32)
            ones_v[r, pl.ds(16, 16)] = jnp.ones((16,), jnp.float32)
            ones_v[r, pl.ds(32, 16)] = jnp.ones((16,), jnp.float32)
            ones_v[r, pl.ds(48, 16)] = jnp.ones((16,), jnp.float32)
            ones_v[r, pl.ds(64, 16)] = jnp.ones((16,), jnp.float32)
            ones_v[r, pl.ds(80, 16)] = jnp.ones((16,), jnp.float32)
            ones_v[r, pl.ds(96, 16)] = jnp.ones((16,), jnp.float32)
            ones_v[r, pl.ds(112, 16)] = jnp.ones((16,), jnp.float32)
            return carry

        lax.fori_loop(0, _SCH, initones, 0)
        plsc.subcore_barrier()

        def body(i, carry):
            base = base0 + i * _SCH
            pltpu.sync_copy(idx_hbm.at[pl.ds(base, _SCH)], idx_v)
            pltpu.sync_copy(ones_v, cnt_sh.at[idx_v], add=True)
            return carry

        lax.fori_loop(0, _PER_W // _SCH, body, 0)
        plsc.subcore_barrier()

        r0 = pl.multiple_of(sid * 640, 8)

        @pl.when(sid < 15)
        def _():
            pltpu.sync_copy(cnt_sh.at[pl.ds(r0, 640)],
                            cnt_hbm.at[cid, pl.ds(r0, 640)])

        @pl.when(sid == 15)
        def _():
            pltpu.sync_copy(cnt_sh.at[pl.ds(9600, 400)],
                            cnt_hbm.at[cid, pl.ds(9600, 400)])

    return k(idx_dst, z128)


_SW = 24


def _p1_body(a1_ref, a2_ref, edge_ref, rbf_ref, ce_ref, wrbf_ref, wfull_ref,
             tg_ref, stats_ref):
    @pl.when(pl.program_id(0) == 0)
    def _():
        stats_ref[...] = jnp.zeros_like(stats_ref)

    wa = wfull_ref[0:_AF, :].astype(jnp.bfloat16)
    wb = wfull_ref[_AF:2 * _AF, :].astype(jnp.bfloat16)
    wc = wfull_ref[2 * _AF:, :]
    nbr = edge_ref[...] * jnp.dot(rbf_ref[...], wrbf_ref[...],
                                  preferred_element_type=jnp.float32)
    tg = (jnp.dot(a1_ref[...].astype(jnp.bfloat16), wa,
                  preferred_element_type=jnp.float32)
          + jnp.dot(a2_ref[...].astype(jnp.bfloat16), wb,
                    preferred_element_type=jnp.float32)
          + jnp.dot(nbr, wc, preferred_element_type=jnp.float32))
    tg_ref[...] = tg.astype(jnp.bfloat16)

    ce = ce_ref[...]
    c_lo = jnp.min(ce)
    c_hi = jnp.max(ce)
    base = pl.multiple_of(jnp.minimum(c_lo - (c_lo % 8), _NC - _SW), 8)
    fastp = (c_hi - base) < _SW

    @pl.when(fastp)
    def _():
        iot = lax.broadcasted_iota(jnp.int32, (1, _SW), 1)
        oh = (ce == base + iot).astype(jnp.float32)
        tgcat = jnp.concatenate(
            [tg, tg * tg, jnp.ones((_BE, _AF), jnp.float32)], axis=1)
        upd = lax.dot_general(oh, tgcat, (((0,), (0,)), ((), ())),
                              preferred_element_type=jnp.float32)
        stats_ref[pl.ds(base, _SW), :] += upd

    @pl.when(jnp.logical_not(fastp))
    def _():
        def crystal_iter(c, carry):
            m = (ce == c).astype(jnp.float32)
            mt = m * tg
            s_row = jnp.sum(mt, axis=0, keepdims=True)
            q_row = jnp.sum(mt * tg, axis=0, keepdims=True)
            n_row = jnp.full((1, _AF), jnp.sum(m), jnp.float32)
            upd = jnp.concatenate([s_row, q_row, n_row], axis=1)
            stats_ref[pl.ds(c, 1), :] += upd
            return carry

        lax.fori_loop(c_lo, c_hi + 1, crystal_iter, 0)


def _edge_pass1(a1, a2, edge, rbf, ce2, W_rbf, W_full):
    return pl.pallas_call(
        _p1_body,
        grid=(_NB,),
        in_specs=[
            pl.BlockSpec((_BE, _AF), lambda i: (i, 0)),
            pl.BlockSpec((_BE, _AF), lambda i: (i, 0)),
            pl.BlockSpec((_BE, _NF), lambda i: (i, 0)),
            pl.BlockSpec((_BE, _NR), lambda i: (i, 0)),
            pl.BlockSpec((_BE, 1), lambda i: (i, 0)),
            pl.BlockSpec((_NR, _NF), lambda i: (0, 0)),
            pl.BlockSpec((2 * _AF + _NF, 2 * _AF), lambda i: (0, 0)),
        ],
        out_specs=[
            pl.BlockSpec((_BE, 2 * _AF), lambda i: (i, 0)),
            pl.BlockSpec((_NC, 5 * _AF), lambda i: (0, 0)),
        ],
        out_shape=[
            jax.ShapeDtypeStruct((_E, 2 * _AF), jnp.bfloat16),
            jax.ShapeDtypeStruct((_NC, 5 * _AF), jnp.float32),
        ],
        compiler_params=pltpu.CompilerParams(
            dimension_semantics=("arbitrary",)),
    )(a1, a2, edge, rbf, ce2, W_rbf, W_full)


def _p2_body(tg_ref, ce_ref, stats_ref, g1_ref, b1_ref, wm_ref, msg_ref):
    tg = tg_ref[...].astype(jnp.float32)
    ce = ce_ref[...]
    c_lo = jnp.min(ce)
    c_hi = jnp.max(ce)
    gamma = g1_ref[...]
    beta = b1_ref[...]
    base = pl.multiple_of(jnp.minimum(c_lo - (c_lo % 8), _NC - _SW), 8)
    fastp = (c_hi - base) < _SW

    def affine_rows(rows):
        n = jnp.maximum(rows[:, 4 * _AF:4 * _AF + 1], 1.0)
        srow = rows[:, 0:2 * _AF] / n
        qrow = rows[:, 2 * _AF:4 * _AF] / n
        var = jnp.maximum(qrow - srow * srow, 0.0)
        a = gamma * lax.rsqrt(var + _EPS)
        b = beta - srow * a
        return a, b

    def fast():
        rows = stats_ref[pl.ds(base, _SW), :]
        a, b = affine_rows(rows)
        iot = lax.broadcasted_iota(jnp.int32, (1, _SW), 1)
        oh = (ce == base + iot).astype(jnp.float32)
        ae = jnp.dot(oh, a, preferred_element_type=jnp.float32)
        be = jnp.dot(oh, b, preferred_element_type=jnp.float32)
        return ae, be

    def slow():
        def crystal_iter(c, carry):
            ae, be = carry
            a, b = affine_rows(stats_ref[pl.ds(c, 1), :])
            m = (ce == c).astype(jnp.float32)
            return ae + m * a, be + m * b

        z = jnp.zeros((_BE, 2 * _AF), jnp.float32)
        return lax.fori_loop(c_lo, c_hi + 1, crystal_iter, (z, z))

    ae, be = lax.cond(fastp, fast, slow)
    tgn = tg * ae + be
    filt = jax.nn.sigmoid(jnp.dot(tgn[:, :_AF], wm_ref[...],
                                  preferred_element_type=jnp.float32))
    core = jnp.maximum(tgn[:, _AF:], 0.0)
    msg_ref[...] = filt * core


def _edge_pass2(tg, ce2, stats, gamma1, beta1, W_mask):
    return pl.pallas_call(
        _p2_body,
        grid=(_NB,),
        in_specs=[
            pl.BlockSpec((_BE, 2 * _AF), lambda i: (i, 0)),
            pl.BlockSpec((_BE, 1), lambda i: (i, 0)),
            pl.BlockSpec((_NC, 5 * _AF), lambda i: (0, 0)),
            pl.BlockSpec((1, 2 * _AF), lambda i: (0, 0)),
            pl.BlockSpec((1, 2 * _AF), lambda i: (0, 0)),
            pl.BlockSpec((_AF, 1), lambda i: (0, 0)),
        ],
        out_specs=pl.BlockSpec((_BE, _AF), lambda i: (i, 0)),
        out_shape=jax.ShapeDtypeStruct((_E, _AF), jnp.float32),
        compiler_params=pltpu.CompilerParams(
            dimension_semantics=("arbitrary",)),
    )(tg, ce2, stats, gamma1, beta1, W_mask)


def _fin_body(acc_ref, cnt_ref, af_ref, ca_ref, g2_ref, b2_ref,
              w10_ref, bb10_ref, w20_ref, bb20_ref,
              w11_ref, bb11_ref, w21_ref, bb21_ref, out_ref):
    acc = acc_ref[0] + acc_ref[1]
    cnt = cnt_ref[0, :, 0:1] + cnt_ref[1, :, 0:1]
    x = acc / jnp.maximum(cnt, 1.0)

    ca = ca_ref[...]
    iot = lax.broadcasted_iota(jnp.int32, (1, _NC), 1)
    onehot = (ca == iot).astype(jnp.float32)
    xcat = jnp.concatenate([x, x * x], axis=1)
    sums = lax.dot_general(onehot, xcat, (((0,), (0,)), ((), ())),
                           preferred_element_type=jnp.float32)
    cntc = jnp.maximum(
        jnp.sum(onehot, axis=0, keepdims=True), 1.0).reshape(_NC, 1)
    mean = sums[:, :_AF] / cntc
    var = jnp.maximum(sums[:, _AF:] / cntc - mean * mean, 0.0)
    ac = g2_ref[...] * lax.rsqrt(var + _EPS)
    bc = b2_ref[...] - mean * ac
    a_e = jnp.dot(onehot, ac, preferred_element_type=jnp.float32)
    b_e = jnp.dot(onehot, bc, preferred_element_type=jnp.float32)
    y = x * a_e + b_e

    h = jnp.maximum(jnp.dot(y, w10_ref[...],
                            preferred_element_type=jnp.float32)
                    + bb10_ref[...], 0.0)
    y = y + jnp.dot(h, w20_ref[...],
                    preferred_element_type=jnp.float32) + bb20_ref[...]
    h = jnp.maximum(jnp.dot(y, w11_ref[...],
                            preferred_element_type=jnp.float32)
                    + bb11_ref[...], 0.0)
    y = y + jnp.dot(h, w21_ref[...],
                    preferred_element_type=jnp.float32) + bb21_ref[...]

    out_ref[...] = 0.7071067811865476 * jnp.maximum(af_ref[...] + y, 0.0)


def _final(acc, cnt, atom_fea, ca2, gamma2, beta2,
           w10, b10, w20, b20, w11, b11, w21, b21):
    full = lambda s: pl.BlockSpec(s, lambda: tuple(0 for _ in s))
    return pl.pallas_call(
        _fin_body,
        in_specs=[
            full((2, _N, _AF)), full((2, _N, _AF)), full((_N, _AF)),
            full((_N, 1)), full((1, _AF)), full((1, _AF)),
            full((_AF, _AF // 2)), full((1, _AF // 2)),
            full((_AF // 2, _AF)), full((1, _AF)),
            full((_AF, _AF // 2)), full((1, _AF // 2)),
            full((_AF // 2, _AF)), full((1, _AF)),
        ],
        out_specs=full((_N, _AF)),
        out_shape=jax.ShapeDtypeStruct((_N, _AF), jnp.float32),
    )(acc, cnt, atom_fea, ca2, gamma2, beta2,
      w10, b10, w20, b20, w11, b11, w21, b21)


def kernel(atom_fea, edge, rbf, nbr_fea_idx, crystal_atom_idx,
           crystal_edge_idx, W_rbf, W_full, W_mask, gamma1, beta1, gamma2,
           beta2, res_W1_0, res_b1_0, res_W2_0, res_b2_0, res_W1_1,
           res_b1_1, res_W2_1, res_b2_1):
    idx_dst = nbr_fea_idx[:, 0]
    idx_src = nbr_fea_idx[:, 1]

    a1, a2 = _sc_gather(atom_fea, idx_dst, idx_src)

    ce2 = crystal_edge_idx.reshape(_E, 1)
    tg, stats = _edge_pass1(a1, a2, edge, rbf, ce2, W_rbf, W_full)
    msg = _edge_pass2(tg, ce2, stats, gamma1.reshape(1, -1),
                      beta1.reshape(1, -1), W_mask)

    z128 = jnp.zeros((_N, _AF), jnp.float32)
    acc = _sc_scatter(msg, idx_dst, z128)
    cnt_at = _sc_count(idx_dst, z128)

    out = _final(acc, cnt_at, atom_fea, crystal_atom_idx.reshape(_N, 1),
                 gamma2.reshape(1, -1), beta2.reshape(1, -1),
                 res_W1_0, res_b1_0.reshape(1, -1),
                 res_W2_0, res_b2_0.reshape(1, -1),
                 res_W1_1, res_b1_1.reshape(1, -1),
                 res_W2_1, res_b2_1.reshape(1, -1))
    return out


# speedup vs baseline: 6.7057x; 1.0552x over previous
import functools

import jax
import jax.numpy as jnp
from jax import lax
from jax.experimental import pallas as pl
from jax.experimental.pallas import tpu as pltpu
from jax.experimental.pallas import tpu_sc as plsc

_N = 10000
_E = 320000
_AF = 128
_NF = 16
_NR = 16
_NC = 256
_EPS = 1e-5

_NW = 32
_PER_W = _E // _NW
_GCH = 80
_SCH = 80
_ROWS_T = _N // 16

_BE = 4000
_NB = _E // _BE


_GC2 = 128
_NGF = _PER_W // _GC2
_GTL = _PER_W - _NGF * _GC2


def _sc_gather(atom_fea, idx_dst, idx_src):
    mesh = plsc.VectorSubcoreMesh(core_axis_name="c", subcore_axis_name="s")

    @functools.partial(
        pl.kernel,
        out_type=(
            jax.ShapeDtypeStruct((_E, _AF), jnp.float32),
            jax.ShapeDtypeStruct((_E, _AF), jnp.float32),
        ),
        mesh=mesh,
        scratch_types=[
            pltpu.VMEM((_GC2,), jnp.int32), pltpu.VMEM((_GC2,), jnp.int32),
            pltpu.VMEM((_GC2,), jnp.int32), pltpu.VMEM((_GC2,), jnp.int32),
            pltpu.VMEM((_GC2, _AF), jnp.float32),
            pltpu.VMEM((_GC2, _AF), jnp.float32),
            pltpu.VMEM((_GC2, _AF), jnp.float32),
            pltpu.VMEM((_GC2, _AF), jnp.float32),
            pltpu.VMEM((_GTL,), jnp.int32), pltpu.VMEM((_GTL,), jnp.int32),
            pltpu.VMEM((_GTL, _AF), jnp.float32),
            pltpu.VMEM((_GTL, _AF), jnp.float32),
            pltpu.SemaphoreType.DMA, pltpu.SemaphoreType.DMA,
            pltpu.SemaphoreType.DMA, pltpu.SemaphoreType.DMA,
            pltpu.SemaphoreType.DMA, pltpu.SemaphoreType.DMA,
            pltpu.SemaphoreType.DMA, pltpu.SemaphoreType.DMA,
        ],
    )
    def k(atom_hbm, i1_hbm, i2_hbm, o1_hbm, o2_hbm,
          i1a, i2a, i1b, i2b, b1a, b2a, b1b, b2b, it1, it2, tb1, tb2,
          s1a, s2a, s1b, s2b, w1a, w2a, w1b, w2b):
        wid = lax.axis_index("s") * 2 + lax.axis_index("c")
        base0 = wid * _PER_W
        bufs = ((i1a, i2a, b1a, b2a, s1a, s2a, w1a, w2a),
                (i1b, i2b, b1b, b2b, s1b, s2b, w1b, w2b))

        def start_gather(p, base):
            i1v, i2v, b1, b2, s1, s2, _, _ = bufs[p]
            pltpu.sync_copy(i1_hbm.at[pl.ds(base, _GC2)], i1v)
            pltpu.sync_copy(i2_hbm.at[pl.ds(base, _GC2)], i2v)
            pltpu.async_copy(atom_hbm.at[i1v], b1, s1)
            pltpu.async_copy(atom_hbm.at[i2v], b2, s2)

        def drain_gather(p, base):
            i1v, i2v, b1, b2, s1, s2, w1, w2 = bufs[p]
            pltpu.make_async_copy(atom_hbm.at[i1v], b1, s1).wait()
            pltpu.make_async_copy(atom_hbm.at[i2v], b2, s2).wait()
            pltpu.async_copy(b1, o1_hbm.at[pl.ds(base, _GC2)], w1)
            pltpu.async_copy(b2, o2_hbm.at[pl.ds(base, _GC2)], w2)

        def wait_wb(p):
            _, _, b1, b2, _, _, w1, w2 = bufs[p]
            pltpu.make_async_copy(b1, o1_hbm.at[pl.ds(base0, _GC2)], w1).wait()
            pltpu.make_async_copy(b2, o2_hbm.at[pl.ds(base0, _GC2)], w2).wait()

        start_gather(0, base0)

        def step(p, j):
            @pl.when(j >= 2)
            def _():
                wait_wb(p)

            start_gather(p, base0 + j * _GC2)
            drain_gather(1 - p, base0 + (j - 1) * _GC2)

        def body(j, carry):
            @pl.when(j % 2 == 1)
            def _():
                step(1, j)

            @pl.when(j % 2 == 0)
            def _():
                step(0, j)

            return carry

        lax.fori_loop(1, _NGF, body, 0)
        drain_gather((_NGF - 1) % 2, base0 + (_NGF - 1) * _GC2)
        wait_wb(0)
        wait_wb(1)

        tbase = base0 + _NGF * _GC2
        pltpu.sync_copy(i1_hbm.at[pl.ds(tbase, _GTL)], it1)
        pltpu.sync_copy(i2_hbm.at[pl.ds(tbase, _GTL)], it2)
        c1 = pltpu.async_copy(atom_hbm.at[it1], tb1, s1a)
        c2 = pltpu.async_copy(atom_hbm.at[it2], tb2, s2a)
        c1.wait()
        c2.wait()
        pltpu.sync_copy(tb1, o1_hbm.at[pl.ds(tbase, _GTL)])
        pltpu.sync_copy(tb2, o2_hbm.at[pl.ds(tbase, _GTL)])

    return k(atom_fea, idx_dst, idx_src)


def _sc_scatter(msg, idx_dst, z128):
    mesh = plsc.VectorSubcoreMesh(core_axis_name="c", subcore_axis_name="s")

    @functools.partial(
        pl.kernel,
        out_type=jax.ShapeDtypeStruct((2, _N, _AF), jnp.float32),
        mesh=mesh,
        scratch_types=[
            pltpu.VMEM((_SCH,), jnp.int32),
            pltpu.VMEM((_SCH,), jnp.int32),
            pltpu.VMEM((_SCH, _AF), jnp.float32),
            pltpu.VMEM((_SCH, _AF), jnp.float32),
            pltpu.VMEM_SHARED((_N, _AF), jnp.float32),
            pltpu.SemaphoreType.DMA, pltpu.SemaphoreType.DMA,
            pltpu.SemaphoreType.DMA, pltpu.SemaphoreType.DMA,
        ],
    )
    def k(msg_hbm, idx_hbm, z128_hbm, acc_hbm, idx_a, idx_b, msg_a, msg_b,
          acc_sh, la, lb, ma, mb):
        cid = lax.axis_index("c")
        sid = lax.axis_index("s")
        wid = sid * 2 + cid
        base0 = wid * _PER_W
        nch = _PER_W // _SCH
        bufs = ((idx_a, msg_a, la, ma), (idx_b, msg_b, lb, mb))

        @pl.when(sid == 0)
        def _():
            pltpu.sync_copy(z128_hbm, acc_sh)

        def start_load(p, base):
            idx_v, msg_v, ls, ms = bufs[p]
            pltpu.async_copy(idx_hbm.at[pl.ds(base, _SCH)], idx_v, ls)
            pltpu.async_copy(msg_hbm.at[pl.ds(base, _SCH)], msg_v, ms)

        def do_scatter(p, base):
            idx_v, msg_v, ls, ms = bufs[p]
            pltpu.make_async_copy(idx_hbm.at[pl.ds(base, _SCH)],
                                  idx_v, ls).wait()
            pltpu.make_async_copy(msg_hbm.at[pl.ds(base, _SCH)],
                                  msg_v, ms).wait()
            pltpu.sync_copy(msg_v, acc_sh.at[idx_v], add=True)

        plsc.subcore_barrier()
        start_load(0, base0)

        def body(j, carry):
            @pl.when(j % 2 == 1)
            def _():
                start_load(1, base0 + j * _SCH)
                do_scatter(0, base0 + (j - 1) * _SCH)

            @pl.when(j % 2 == 0)
            def _():
                start_load(0, base0 + j * _SCH)
                do_scatter(1, base0 + (j - 1) * _SCH)

            return carry

        lax.fori_loop(1, nch, body, 0)
        do_scatter((nch - 1) % 2, base0 + (nch - 1) * _SCH)
        plsc.subcore_barrier()

        r0 = pl.multiple_of(sid * 640, 8)

        @pl.when(sid < 15)
        def _():
            pltpu.sync_copy(acc_sh.at[pl.ds(r0, 640)],
                            acc_hbm.at[cid, pl.ds(r0, 640)])

        @pl.when(sid == 15)
        def _():
            pltpu.sync_copy(acc_sh.at[pl.ds(9600, 400)],
                            acc_hbm.at[cid, pl.ds(9600, 400)])

    return k(msg, idx_dst, z128)


def _sc_count(idx_dst, z128):
    mesh = plsc.VectorSubcoreMesh(core_axis_name="c", subcore_axis_name="s")

    @functools.partial(
        pl.kernel,
        out_type=jax.ShapeDtypeStruct((2, _N, _AF), jnp.float32),
        mesh=mesh,
        scratch_types=[
            pltpu.VMEM((_SCH,), jnp.int32),
            pltpu.VMEM((_SCH, _AF), jnp.float32),
            pltpu.VMEM_SHARED((_N, _AF), jnp.float32),
        ],
    )
    def k(idx_hbm, z128_hbm, cnt_hbm, idx_v, ones_v, cnt_sh):
        cid = lax.axis_index("c")
        sid = lax.axis_index("s")
        wid = sid * 2 + cid
        base0 = wid * _PER_W

        @pl.when(sid == 0)
        def _():
            pltpu.sync_copy(z128_hbm, cnt_sh)

        def initones(r, carry):
            ones_v[r, pl.ds(0, 16)] = jnp.ones((16,), jnp.float32)
            ones_v[r, pl.ds(16, 16)] = jnp.ones((16,), jnp.float32)
            ones_v[r, pl.ds(32, 16)] = jnp.ones((16,), jnp.float32)
            ones_v[r, pl.ds(48, 16)] = jnp.ones((16,), jnp.float32)
            ones_v[r, pl.ds(64, 16)] = jnp.ones((16,), jnp.float32)
            ones_v[r, pl.ds(80, 16)] = jnp.ones((16,), jnp.float32)
            ones_v[r, pl.ds(96, 16)] = jnp.ones((16,), jnp.float32)
            ones_v[r, pl.ds(112, 16)] = jnp.ones((16,), jnp.float32)
            return carry

        lax.fori_loop(0, _SCH, initones, 0)
        plsc.subcore_barrier()

        def body(i, carry):
            base = base0 + i * _SCH
            pltpu.sync_copy(idx_hbm.at[pl.ds(base, _SCH)], idx_v)
            pltpu.sync_copy(ones_v, cnt_sh.at[idx_v], add=True)
            return carry

        lax.fori_loop(0, _PER_W // _SCH, body, 0)
        plsc.subcore_barrier()

        r0 = pl.multiple_of(sid * 640, 8)

        @pl.when(sid < 15)
        def _():
            pltpu.sync_copy(cnt_sh.at[pl.ds(r0, 640)],
                            cnt_hbm.at[cid, pl.ds(r0, 640)])

        @pl.when(sid == 15)
        def _():
            pltpu.sync_copy(cnt_sh.at[pl.ds(9600, 400)],
                            cnt_hbm.at[cid, pl.ds(9600, 400)])

    return k(idx_dst, z128)


_SW = 24


def _p1_body(a1_ref, a2_ref, edge_ref, rbf_ref, ce_ref, wrbf_ref, wfull_ref,
             tg_ref, stats_ref):
    @pl.when(pl.program_id(0) == 0)
    def _():
        stats_ref[...] = jnp.zeros_like(stats_ref)

    wa = wfull_ref[0:_AF, :].astype(jnp.bfloat16)
    wb = wfull_ref[_AF:2 * _AF, :].astype(jnp.bfloat16)
    wc = wfull_ref[2 * _AF:, :]
    nbr = edge_ref[...] * jnp.dot(rbf_ref[...], wrbf_ref[...],
                                  preferred_element_type=jnp.float32)
    tg = (jnp.dot(a1_ref[...].astype(jnp.bfloat16), wa,
                  preferred_element_type=jnp.float32)
          + jnp.dot(a2_ref[...].astype(jnp.bfloat16), wb,
                    preferred_element_type=jnp.float32)
          + jnp.dot(nbr, wc, preferred_element_type=jnp.float32))
    tg_ref[...] = tg.astype(jnp.bfloat16)

    ce = ce_ref[...]
    c_lo = jnp.min(ce)
    c_hi = jnp.max(ce)
    base = pl.multiple_of(jnp.minimum(c_lo - (c_lo % 8), _NC - _SW), 8)
    fastp = (c_hi - base) < _SW

    @pl.when(fastp)
    def _():
        iot = lax.broadcasted_iota(jnp.int32, (1, _SW), 1)
        oh = (ce == base + iot).astype(jnp.float32)
        tgcat = jnp.concatenate(
            [tg, tg * tg, jnp.ones((_BE, _AF), jnp.float32)], axis=1)
        upd = lax.dot_general(oh, tgcat, (((0,), (0,)), ((), ())),
                              preferred_element_type=jnp.float32)
        stats_ref[pl.ds(base, _SW), :] += upd

    @pl.when(jnp.logical_not(fastp))
    def _():
        def crystal_iter(c, carry):
            m = (ce == c).astype(jnp.float32)
            mt = m * tg
            s_row = jnp.sum(mt, axis=0, keepdims=True)
            q_row = jnp.sum(mt * tg, axis=0, keepdims=True)
            n_row = jnp.full((1, _AF), jnp.sum(m), jnp.float32)
            upd = jnp.concatenate([s_row, q_row, n_row], axis=1)
            stats_ref[pl.ds(c, 1), :] += upd
            return carry

        lax.fori_loop(c_lo, c_hi + 1, crystal_iter, 0)


def _edge_pass1(a1, a2, edge, rbf, ce2, W_rbf, W_full):
    return pl.pallas_call(
        _p1_body,
        grid=(_NB,),
        in_specs=[
            pl.BlockSpec((_BE, _AF), lambda i: (i, 0)),
            pl.BlockSpec((_BE, _AF), lambda i: (i, 0)),
            pl.BlockSpec((_BE, _NF), lambda i: (i, 0)),
            pl.BlockSpec((_BE, _NR), lambda i: (i, 0)),
            pl.BlockSpec((_BE, 1), lambda i: (i, 0)),
            pl.BlockSpec((_NR, _NF), lambda i: (0, 0)),
            pl.BlockSpec((2 * _AF + _NF, 2 * _AF), lambda i: (0, 0)),
        ],
        out_specs=[
            pl.BlockSpec((_BE, 2 * _AF), lambda i: (i, 0)),
            pl.BlockSpec((_NC, 5 * _AF), lambda i: (0, 0)),
        ],
        out_shape=[
            jax.ShapeDtypeStruct((_E, 2 * _AF), jnp.bfloat16),
            jax.ShapeDtypeStruct((_NC, 5 * _AF), jnp.float32),
        ],
        compiler_params=pltpu.CompilerParams(
            dimension_semantics=("arbitrary",)),
    )(a1, a2, edge, rbf, ce2, W_rbf, W_full)


def _p2_body(tg_ref, ce_ref, stats_ref, g1_ref, b1_ref, wm_ref, msg_ref):
    tg = tg_ref[...].astype(jnp.float32)
    ce = ce_ref[...]
    c_lo = jnp.min(ce)
    c_hi = jnp.max(ce)
    gamma = g1_ref[...]
    beta = b1_ref[...]
    base = pl.multiple_of(jnp.minimum(c_lo - (c_lo % 8), _NC - _SW), 8)
    fastp = (c_hi - base) < _SW

    def affine_rows(rows):
        n = jnp.maximum(rows[:, 4 * _AF:4 * _AF + 1], 1.0)
        srow = rows[:, 0:2 * _AF] / n
        qrow = rows[:, 2 * _AF:4 * _AF] / n
        var = jnp.maximum(qrow - srow * srow, 0.0)
        a = gamma * lax.rsqrt(var + _EPS)
        b = beta - srow * a
        return a, b

    def fast():
        rows = stats_ref[pl.ds(base, _SW), :]
        a, b = affine_rows(rows)
        iot = lax.broadcasted_iota(jnp.int32, (1, _SW), 1)
        oh = (ce == base + iot).astype(jnp.float32)
        ae = jnp.dot(oh, a, preferred_element_type=jnp.float32)
        be = jnp.dot(oh, b, preferred_element_type=jnp.float32)
        return ae, be

    def slow():
        def crystal_iter(c, carry):
            ae, be = carry
            a, b = affine_rows(stats_ref[pl.ds(c, 1), :])
            m = (ce == c).astype(jnp.float32)
            return ae + m * a, be + m * b

        z = jnp.zeros((_BE, 2 * _AF), jnp.float32)
        return lax.fori_loop(c_lo, c_hi + 1, crystal_iter, (z, z))

    ae, be = lax.cond(fastp, fast, slow)
    tgn = tg * ae + be
    filt = jax.nn.sigmoid(jnp.dot(tgn[:, :_AF], wm_ref[...],
                                  preferred_element_type=jnp.float32))
    core = jnp.maximum(tgn[:, _AF:], 0.0)
    msg_ref[...] = filt * core


def _edge_pass2(tg, ce2, stats, gamma1, beta1, W_mask):
    return pl.pallas_call(
        _p2_body,
        grid=(_NB,),
        in_specs=[
            pl.BlockSpec((_BE, 2 * _AF), lambda i: (i, 0)),
            pl.BlockSpec((_BE, 1), lambda i: (i, 0)),
            pl.BlockSpec((_NC, 5 * _AF), lambda i: (0, 0)),
            pl.BlockSpec((1, 2 * _AF), lambda i: (0, 0)),
            pl.BlockSpec((1, 2 * _AF), lambda i: (0, 0)),
            pl.BlockSpec((_AF, 1), lambda i: (0, 0)),
        ],
        out_specs=pl.BlockSpec((_BE, _AF), lambda i: (i, 0)),
        out_shape=jax.ShapeDtypeStruct((_E, _AF), jnp.float32),
        compiler_params=pltpu.CompilerParams(
            dimension_semantics=("arbitrary",)),
    )(tg, ce2, stats, gamma1, beta1, W_mask)


def _fin_body(acc_ref, cnt_ref, af_ref, ca_ref, g2_ref, b2_ref,
              w10_ref, bb10_ref, w20_ref, bb20_ref,
              w11_ref, bb11_ref, w21_ref, bb21_ref, out_ref):
    acc = acc_ref[0] + acc_ref[1]
    cnt = cnt_ref[0, :, 0:1] + cnt_ref[1, :, 0:1]
    x = acc / jnp.maximum(cnt, 1.0)

    ca = ca_ref[...]
    iot = lax.broadcasted_iota(jnp.int32, (1, _NC), 1)
    onehot = (ca == iot).astype(jnp.float32)
    xcat = jnp.concatenate([x, x * x], axis=1)
    sums = lax.dot_general(onehot, xcat, (((0,), (0,)), ((), ())),
                           preferred_element_type=jnp.float32)
    cntc = jnp.maximum(
        jnp.sum(onehot, axis=0, keepdims=True), 1.0).reshape(_NC, 1)
    mean = sums[:, :_AF] / cntc
    var = jnp.maximum(sums[:, _AF:] / cntc - mean * mean, 0.0)
    ac = g2_ref[...] * lax.rsqrt(var + _EPS)
    bc = b2_ref[...] - mean * ac
    a_e = jnp.dot(onehot, ac, preferred_element_type=jnp.float32)
    b_e = jnp.dot(onehot, bc, preferred_element_type=jnp.float32)
    y = x * a_e + b_e

    h = jnp.maximum(jnp.dot(y, w10_ref[...],
                            preferred_element_type=jnp.float32)
                    + bb10_ref[...], 0.0)
    y = y + jnp.dot(h, w20_ref[...],
                    preferred_element_type=jnp.float32) + bb20_ref[...]
    h = jnp.maximum(jnp.dot(y, w11_ref[...],
                            preferred_element_type=jnp.float32)
                    + bb11_ref[...], 0.0)
    y = y + jnp.dot(h, w21_ref[...],
                    preferred_element_type=jnp.float32) + bb21_ref[...]

    out_ref[...] = 0.7071067811865476 * jnp.maximum(af_ref[...] + y, 0.0)


def _final(acc, cnt, atom_fea, ca2, gamma2, beta2,
           w10, b10, w20, b20, w11, b11, w21, b21):
    full = lambda s: pl.BlockSpec(s, lambda: tuple(0 for _ in s))
    return pl.pallas_call(
        _fin_body,
        in_specs=[
            full((2, _N, _AF)), full((2, _N, _AF)), full((_N, _AF)),
            full((_N, 1)), full((1, _AF)), full((1, _AF)),
            full((_AF, _AF // 2)), full((1, _AF // 2)),
            full((_AF // 2, _AF)), full((1, _AF)),
            full((_AF, _AF // 2)), full((1, _AF // 2)),
            full((_AF // 2, _AF)), full((1, _AF)),
        ],
        out_specs=full((_N, _AF)),
        out_shape=jax.ShapeDtypeStruct((_N, _AF), jnp.float32),
    )(acc, cnt, atom_fea, ca2, gamma2, beta2,
      w10, b10, w20, b20, w11, b11, w21, b21)


def kernel(atom_fea, edge, rbf, nbr_fea_idx, crystal_atom_idx,
           crystal_edge_idx, W_rbf, W_full, W_mask, gamma1, beta1, gamma2,
           beta2, res_W1_0, res_b1_0, res_W2_0, res_b2_0, res_W1_1,
           res_b1_1, res_W2_1, res_b2_1):
    idx_dst = nbr_fea_idx[:, 0]
    idx_src = nbr_fea_idx[:, 1]

    a1, a2 = _sc_gather(atom_fea, idx_dst, idx_src)

    ce2 = crystal_edge_idx.reshape(_E, 1)
    tg, stats = _edge_pass1(a1, a2, edge, rbf, ce2, W_rbf, W_full)
    msg = _edge_pass2(tg, ce2, stats, gamma1.reshape(1, -1),
                      beta1.reshape(1, -1), W_mask)

    z128 = jnp.zeros((_N, _AF), jnp.float32)
    acc = _sc_scatter(msg, idx_dst, z128)
    cnt_at = _sc_count(idx_dst, z128)

    out = _final(acc, cnt_at, atom_fea, crystal_atom_idx.reshape(_N, 1),
                 gamma2.reshape(1, -1), beta2.reshape(1, -1),
                 res_W1_0, res_b1_0.reshape(1, -1),
                 res_W2_0, res_b2_0.reshape(1, -1),
                 res_W1_1, res_b1_1.reshape(1, -1),
                 res_W2_1, res_b2_1.reshape(1, -1))
    return out


# speedup vs baseline: 6.7460x; 1.0060x over previous
import functools

import jax
import jax.numpy as jnp
from jax import lax
from jax.experimental import pallas as pl
from jax.experimental.pallas import tpu as pltpu
from jax.experimental.pallas import tpu_sc as plsc

_N = 10000
_E = 320000
_AF = 128
_NF = 16
_NR = 16
_NC = 256
_EPS = 1e-5

_NW = 32
_PER_W = _E // _NW
_GCH = 80
_SCH = 80
_ROWS_T = _N // 16

_BE = 5000
_NB = _E // _BE


_GC2 = 128
_NGF = _PER_W // _GC2
_GTL = _PER_W - _NGF * _GC2


def _sc_gather(atom_fea, idx_dst, idx_src):
    mesh = plsc.VectorSubcoreMesh(core_axis_name="c", subcore_axis_name="s")

    @functools.partial(
        pl.kernel,
        out_type=(
            jax.ShapeDtypeStruct((_E, _AF), jnp.float32),
            jax.ShapeDtypeStruct((_E, _AF), jnp.float32),
        ),
        mesh=mesh,
        scratch_types=[
            pltpu.VMEM((_GC2,), jnp.int32), pltpu.VMEM((_GC2,), jnp.int32),
            pltpu.VMEM((_GC2,), jnp.int32), pltpu.VMEM((_GC2,), jnp.int32),
            pltpu.VMEM((_GC2, _AF), jnp.float32),
            pltpu.VMEM((_GC2, _AF), jnp.float32),
            pltpu.VMEM((_GC2, _AF), jnp.float32),
            pltpu.VMEM((_GC2, _AF), jnp.float32),
            pltpu.VMEM((_GTL,), jnp.int32), pltpu.VMEM((_GTL,), jnp.int32),
            pltpu.VMEM((_GTL, _AF), jnp.float32),
            pltpu.VMEM((_GTL, _AF), jnp.float32),
            pltpu.SemaphoreType.DMA, pltpu.SemaphoreType.DMA,
            pltpu.SemaphoreType.DMA, pltpu.SemaphoreType.DMA,
            pltpu.SemaphoreType.DMA, pltpu.SemaphoreType.DMA,
            pltpu.SemaphoreType.DMA, pltpu.SemaphoreType.DMA,
        ],
    )
    def k(atom_hbm, i1_hbm, i2_hbm, o1_hbm, o2_hbm,
          i1a, i2a, i1b, i2b, b1a, b2a, b1b, b2b, it1, it2, tb1, tb2,
          s1a, s2a, s1b, s2b, w1a, w2a, w1b, w2b):
        wid = lax.axis_index("s") * 2 + lax.axis_index("c")
        base0 = wid * _PER_W
        bufs = ((i1a, i2a, b1a, b2a, s1a, s2a, w1a, w2a),
                (i1b, i2b, b1b, b2b, s1b, s2b, w1b, w2b))

        def start_gather(p, base):
            i1v, i2v, b1, b2, s1, s2, _, _ = bufs[p]
            pltpu.sync_copy(i1_hbm.at[pl.ds(base, _GC2)], i1v)
            pltpu.sync_copy(i2_hbm.at[pl.ds(base, _GC2)], i2v)
            pltpu.async_copy(atom_hbm.at[i1v], b1, s1)
            pltpu.async_copy(atom_hbm.at[i2v], b2, s2)

        def drain_gather(p, base):
            i1v, i2v, b1, b2, s1, s2, w1, w2 = bufs[p]
            pltpu.make_async_copy(atom_hbm.at[i1v], b1, s1).wait()
            pltpu.make_async_copy(atom_hbm.at[i2v], b2, s2).wait()
            pltpu.async_copy(b1, o1_hbm.at[pl.ds(base, _GC2)], w1)
            pltpu.async_copy(b2, o2_hbm.at[pl.ds(base, _GC2)], w2)

        def wait_wb(p):
            _, _, b1, b2, _, _, w1, w2 = bufs[p]
            pltpu.make_async_copy(b1, o1_hbm.at[pl.ds(base0, _GC2)], w1).wait()
            pltpu.make_async_copy(b2, o2_hbm.at[pl.ds(base0, _GC2)], w2).wait()

        start_gather(0, base0)

        def step(p, j):
            @pl.when(j >= 2)
            def _():
                wait_wb(p)

            start_gather(p, base0 + j * _GC2)
            drain_gather(1 - p, base0 + (j - 1) * _GC2)

        def body(j, carry):
            @pl.when(j % 2 == 1)
            def _():
                step(1, j)

            @pl.when(j % 2 == 0)
            def _():
                step(0, j)

            return carry

        lax.fori_loop(1, _NGF, body, 0)
        drain_gather((_NGF - 1) % 2, base0 + (_NGF - 1) * _GC2)
        wait_wb(0)
        wait_wb(1)

        tbase = base0 + _NGF * _GC2
        pltpu.sync_copy(i1_hbm.at[pl.ds(tbase, _GTL)], it1)
        pltpu.sync_copy(i2_hbm.at[pl.ds(tbase, _GTL)], it2)
        c1 = pltpu.async_copy(atom_hbm.at[it1], tb1, s1a)
        c2 = pltpu.async_copy(atom_hbm.at[it2], tb2, s2a)
        c1.wait()
        c2.wait()
        pltpu.sync_copy(tb1, o1_hbm.at[pl.ds(tbase, _GTL)])
        pltpu.sync_copy(tb2, o2_hbm.at[pl.ds(tbase, _GTL)])

    return k(atom_fea, idx_dst, idx_src)


def _sc_scatter(msg, idx_dst, z128):
    mesh = plsc.VectorSubcoreMesh(core_axis_name="c", subcore_axis_name="s")

    @functools.partial(
        pl.kernel,
        out_type=jax.ShapeDtypeStruct((2, _N, _AF), jnp.float32),
        mesh=mesh,
        scratch_types=[
            pltpu.VMEM((_SCH,), jnp.int32),
            pltpu.VMEM((_SCH,), jnp.int32),
            pltpu.VMEM((_SCH, _AF), jnp.float32),
            pltpu.VMEM((_SCH, _AF), jnp.float32),
            pltpu.VMEM_SHARED((_N, _AF), jnp.float32),
            pltpu.SemaphoreType.DMA, pltpu.SemaphoreType.DMA,
            pltpu.SemaphoreType.DMA, pltpu.SemaphoreType.DMA,
        ],
    )
    def k(msg_hbm, idx_hbm, z128_hbm, acc_hbm, idx_a, idx_b, msg_a, msg_b,
          acc_sh, la, lb, ma, mb):
        cid = lax.axis_index("c")
        sid = lax.axis_index("s")
        wid = sid * 2 + cid
        base0 = wid * _PER_W
        nch = _PER_W // _SCH
        bufs = ((idx_a, msg_a, la, ma), (idx_b, msg_b, lb, mb))

        @pl.when(sid == 0)
        def _():
            pltpu.sync_copy(z128_hbm, acc_sh)

        def start_load(p, base):
            idx_v, msg_v, ls, ms = bufs[p]
            pltpu.async_copy(idx_hbm.at[pl.ds(base, _SCH)], idx_v, ls)
            pltpu.async_copy(msg_hbm.at[pl.ds(base, _SCH)], msg_v, ms)

        def do_scatter(p, base):
            idx_v, msg_v, ls, ms = bufs[p]
            pltpu.make_async_copy(idx_hbm.at[pl.ds(base, _SCH)],
                                  idx_v, ls).wait()
            pltpu.make_async_copy(msg_hbm.at[pl.ds(base, _SCH)],
                                  msg_v, ms).wait()
            pltpu.sync_copy(msg_v, acc_sh.at[idx_v], add=True)

        plsc.subcore_barrier()
        start_load(0, base0)

        def body(j, carry):
            @pl.when(j % 2 == 1)
            def _():
                start_load(1, base0 + j * _SCH)
                do_scatter(0, base0 + (j - 1) * _SCH)

            @pl.when(j % 2 == 0)
            def _():
                start_load(0, base0 + j * _SCH)
                do_scatter(1, base0 + (j - 1) * _SCH)

            return carry

        lax.fori_loop(1, nch, body, 0)
        do_scatter((nch - 1) % 2, base0 + (nch - 1) * _SCH)
        plsc.subcore_barrier()

        r0 = pl.multiple_of(sid * 640, 8)

        @pl.when(sid < 15)
        def _():
            pltpu.sync_copy(acc_sh.at[pl.ds(r0, 640)],
                            acc_hbm.at[cid, pl.ds(r0, 640)])

        @pl.when(sid == 15)
        def _():
            pltpu.sync_copy(acc_sh.at[pl.ds(9600, 400)],
                            acc_hbm.at[cid, pl.ds(9600, 400)])

    return k(msg, idx_dst, z128)


def _sc_count(idx_dst, z128):
    mesh = plsc.VectorSubcoreMesh(core_axis_name="c", subcore_axis_name="s")

    @functools.partial(
        pl.kernel,
        out_type=jax.ShapeDtypeStruct((2, _N, _AF), jnp.float32),
        mesh=mesh,
        scratch_types=[
            pltpu.VMEM((_SCH,), jnp.int32),
            pltpu.VMEM((_SCH, _AF), jnp.float32),
            pltpu.VMEM_SHARED((_N, _AF), jnp.float32),
        ],
    )
    def k(idx_hbm, z128_hbm, cnt_hbm, idx_v, ones_v, cnt_sh):
        cid = lax.axis_index("c")
        sid = lax.axis_index("s")
        wid = sid * 2 + cid
        base0 = wid * _PER_W

        @pl.when(sid == 0)
        def _():
            pltpu.sync_copy(z128_hbm, cnt_sh)

        def initones(r, carry):
            ones_v[r, pl.ds(0, 16)] = jnp.ones((16,), jnp.float32)
            ones_v[r, pl.ds(16, 16)] = jnp.ones((16,), jnp.float32)
            ones_v[r, pl.ds(32, 16)] = jnp.ones((16,), jnp.float32)
            ones_v[r, pl.ds(48, 16)] = jnp.ones((16,), jnp.float32)
            ones_v[r, pl.ds(64, 16)] = jnp.ones((16,), jnp.float32)
            ones_v[r, pl.ds(80, 16)] = jnp.ones((16,), jnp.float32)
            ones_v[r, pl.ds(96, 16)] = jnp.ones((16,), jnp.float32)
            ones_v[r, pl.ds(112, 16)] = jnp.ones((16,), jnp.float32)
            return carry

        lax.fori_loop(0, _SCH, initones, 0)
        plsc.subcore_barrier()

        def body(i, carry):
            base = base0 + i * _SCH
            pltpu.sync_copy(idx_hbm.at[pl.ds(base, _SCH)], idx_v)
            pltpu.sync_copy(ones_v, cnt_sh.at[idx_v], add=True)
            return carry

        lax.fori_loop(0, _PER_W // _SCH, body, 0)
        plsc.subcore_barrier()

        r0 = pl.multiple_of(sid * 640, 8)

        @pl.when(sid < 15)
        def _():
            pltpu.sync_copy(cnt_sh.at[pl.ds(r0, 640)],
                            cnt_hbm.at[cid, pl.ds(r0, 640)])

        @pl.when(sid == 15)
        def _():
            pltpu.sync_copy(cnt_sh.at[pl.ds(9600, 400)],
                            cnt_hbm.at[cid, pl.ds(9600, 400)])

    return k(idx_dst, z128)


_SW = 24


def _p1_body(a1_ref, a2_ref, edge_ref, rbf_ref, ce_ref, wrbf_ref, wfull_ref,
             tg_ref, stats_ref):
    @pl.when(pl.program_id(0) == 0)
    def _():
        stats_ref[...] = jnp.zeros_like(stats_ref)

    wa = wfull_ref[0:_AF, :].astype(jnp.bfloat16)
    wb = wfull_ref[_AF:2 * _AF, :].astype(jnp.bfloat16)
    wc = wfull_ref[2 * _AF:, :]
    nbr = edge_ref[...] * jnp.dot(rbf_ref[...], wrbf_ref[...],
                                  preferred_element_type=jnp.float32)
    tg = (jnp.dot(a1_ref[...].astype(jnp.bfloat16), wa,
                  preferred_element_type=jnp.float32)
          + jnp.dot(a2_ref[...].astype(jnp.bfloat16), wb,
                    preferred_element_type=jnp.float32)
          + jnp.dot(nbr, wc, preferred_element_type=jnp.float32))
    tg_ref[...] = tg.astype(jnp.bfloat16)

    ce = ce_ref[...]
    c_lo = jnp.min(ce)
    c_hi = jnp.max(ce)
    base = pl.multiple_of(jnp.minimum(c_lo - (c_lo % 8), _NC - _SW), 8)
    fastp = (c_hi - base) < _SW

    @pl.when(fastp)
    def _():
        iot = lax.broadcasted_iota(jnp.int32, (1, _SW), 1)
        oh = (ce == base + iot).astype(jnp.float32)
        tgcat = jnp.concatenate(
            [tg, tg * tg, jnp.ones((_BE, _AF), jnp.float32)], axis=1)
        upd = lax.dot_general(oh, tgcat, (((0,), (0,)), ((), ())),
                              preferred_element_type=jnp.float32)
        stats_ref[pl.ds(base, _SW), :] += upd

    @pl.when(jnp.logical_not(fastp))
    def _():
        def crystal_iter(c, carry):
            m = (ce == c).astype(jnp.float32)
            mt = m * tg
            s_row = jnp.sum(mt, axis=0, keepdims=True)
            q_row = jnp.sum(mt * tg, axis=0, keepdims=True)
            n_row = jnp.full((1, _AF), jnp.sum(m), jnp.float32)
            upd = jnp.concatenate([s_row, q_row, n_row], axis=1)
            stats_ref[pl.ds(c, 1), :] += upd
            return carry

        lax.fori_loop(c_lo, c_hi + 1, crystal_iter, 0)


def _edge_pass1(a1, a2, edge, rbf, ce2, W_rbf, W_full):
    return pl.pallas_call(
        _p1_body,
        grid=(_NB,),
        in_specs=[
            pl.BlockSpec((_BE, _AF), lambda i: (i, 0)),
            pl.BlockSpec((_BE, _AF), lambda i: (i, 0)),
            pl.BlockSpec((_BE, _NF), lambda i: (i, 0)),
            pl.BlockSpec((_BE, _NR), lambda i: (i, 0)),
            pl.BlockSpec((_BE, 1), lambda i: (i, 0)),
            pl.BlockSpec((_NR, _NF), lambda i: (0, 0)),
            pl.BlockSpec((2 * _AF + _NF, 2 * _AF), lambda i: (0, 0)),
        ],
        out_specs=[
            pl.BlockSpec((_BE, 2 * _AF), lambda i: (i, 0)),
            pl.BlockSpec((_NC, 5 * _AF), lambda i: (0, 0)),
        ],
        out_shape=[
            jax.ShapeDtypeStruct((_E, 2 * _AF), jnp.bfloat16),
            jax.ShapeDtypeStruct((_NC, 5 * _AF), jnp.float32),
        ],
        compiler_params=pltpu.CompilerParams(
            dimension_semantics=("arbitrary",)),
    )(a1, a2, edge, rbf, ce2, W_rbf, W_full)


def _p2_body(tg_ref, ce_ref, stats_ref, g1_ref, b1_ref, wm_ref, msg_ref):
    tg = tg_ref[...].astype(jnp.float32)
    ce = ce_ref[...]
    c_lo = jnp.min(ce)
    c_hi = jnp.max(ce)
    gamma = g1_ref[...]
    beta = b1_ref[...]
    base = pl.multiple_of(jnp.minimum(c_lo - (c_lo % 8), _NC - _SW), 8)
    fastp = (c_hi - base) < _SW

    def affine_rows(rows):
        n = jnp.maximum(rows[:, 4 * _AF:4 * _AF + 1], 1.0)
        srow = rows[:, 0:2 * _AF] / n
        qrow = rows[:, 2 * _AF:4 * _AF] / n
        var = jnp.maximum(qrow - srow * srow, 0.0)
        a = gamma * lax.rsqrt(var + _EPS)
        b = beta - srow * a
        return a, b

    def fast():
        rows = stats_ref[pl.ds(base, _SW), :]
        a, b = affine_rows(rows)
        iot = lax.broadcasted_iota(jnp.int32, (1, _SW), 1)
        oh = (ce == base + iot).astype(jnp.float32)
        ae = jnp.dot(oh, a, preferred_element_type=jnp.float32)
        be = jnp.dot(oh, b, preferred_element_type=jnp.float32)
        return ae, be

    def slow():
        def crystal_iter(c, carry):
            ae, be = carry
            a, b = affine_rows(stats_ref[pl.ds(c, 1), :])
            m = (ce == c).astype(jnp.float32)
            return ae + m * a, be + m * b

        z = jnp.zeros((_BE, 2 * _AF), jnp.float32)
        return lax.fori_loop(c_lo, c_hi + 1, crystal_iter, (z, z))

    ae, be = lax.cond(fastp, fast, slow)
    tgn = tg * ae + be
    filt = jax.nn.sigmoid(jnp.dot(tgn[:, :_AF], wm_ref[...],
                                  preferred_element_type=jnp.float32))
    core = jnp.maximum(tgn[:, _AF:], 0.0)
    msg_ref[...] = filt * core


def _edge_pass2(tg, ce2, stats, gamma1, beta1, W_mask):
    return pl.pallas_call(
        _p2_body,
        grid=(_NB,),
        in_specs=[
            pl.BlockSpec((_BE, 2 * _AF), lambda i: (i, 0)),
            pl.BlockSpec((_BE, 1), lambda i: (i, 0)),
            pl.BlockSpec((_NC, 5 * _AF), lambda i: (0, 0)),
            pl.BlockSpec((1, 2 * _AF), lambda i: (0, 0)),
            pl.BlockSpec((1, 2 * _AF), lambda i: (0, 0)),
            pl.BlockSpec((_AF, 1), lambda i: (0, 0)),
        ],
        out_specs=pl.BlockSpec((_BE, _AF), lambda i: (i, 0)),
        out_shape=jax.ShapeDtypeStruct((_E, _AF), jnp.float32),
        compiler_params=pltpu.CompilerParams(
            dimension_semantics=("arbitrary",)),
    )(tg, ce2, stats, gamma1, beta1, W_mask)


def _fin_body(acc_ref, cnt_ref, af_ref, ca_ref, g2_ref, b2_ref,
              w10_ref, bb10_ref, w20_ref, bb20_ref,
              w11_ref, bb11_ref, w21_ref, bb21_ref, out_ref):
    acc = acc_ref[0] + acc_ref[1]
    cnt = cnt_ref[0, :, 0:1] + cnt_ref[1, :, 0:1]
    x = acc / jnp.maximum(cnt, 1.0)

    ca = ca_ref[...]
    iot = lax.broadcasted_iota(jnp.int32, (1, _NC), 1)
    onehot = (ca == iot).astype(jnp.float32)
    xcat = jnp.concatenate([x, x * x], axis=1)
    sums = lax.dot_general(onehot, xcat, (((0,), (0,)), ((), ())),
                           preferred_element_type=jnp.float32)
    cntc = jnp.maximum(
        jnp.sum(onehot, axis=0, keepdims=True), 1.0).reshape(_NC, 1)
    mean = sums[:, :_AF] / cntc
    var = jnp.maximum(sums[:, _AF:] / cntc - mean * mean, 0.0)
    ac = g2_ref[...] * lax.rsqrt(var + _EPS)
    bc = b2_ref[...] - mean * ac
    a_e = jnp.dot(onehot, ac, preferred_element_type=jnp.float32)
    b_e = jnp.dot(onehot, bc, preferred_element_type=jnp.float32)
    y = x * a_e + b_e

    h = jnp.maximum(jnp.dot(y, w10_ref[...],
                            preferred_element_type=jnp.float32)
                    + bb10_ref[...], 0.0)
    y = y + jnp.dot(h, w20_ref[...],
                    preferred_element_type=jnp.float32) + bb20_ref[...]
    h = jnp.maximum(jnp.dot(y, w11_ref[...],
                            preferred_element_type=jnp.float32)
                    + bb11_ref[...], 0.0)
    y = y + jnp.dot(h, w21_ref[...],
                    preferred_element_type=jnp.float32) + bb21_ref[...]

    out_ref[...] = 0.7071067811865476 * jnp.maximum(af_ref[...] + y, 0.0)


def _final(acc, cnt, atom_fea, ca2, gamma2, beta2,
           w10, b10, w20, b20, w11, b11, w21, b21):
    full = lambda s: pl.BlockSpec(s, lambda: tuple(0 for _ in s))
    return pl.pallas_call(
        _fin_body,
        in_specs=[
            full((2, _N, _AF)), full((2, _N, _AF)), full((_N, _AF)),
            full((_N, 1)), full((1, _AF)), full((1, _AF)),
            full((_AF, _AF // 2)), full((1, _AF // 2)),
            full((_AF // 2, _AF)), full((1, _AF)),
            full((_AF, _AF // 2)), full((1, _AF // 2)),
            full((_AF // 2, _AF)), full((1, _AF)),
        ],
        out_specs=full((_N, _AF)),
        out_shape=jax.ShapeDtypeStruct((_N, _AF), jnp.float32),
    )(acc, cnt, atom_fea, ca2, gamma2, beta2,
      w10, b10, w20, b20, w11, b11, w21, b21)


def kernel(atom_fea, edge, rbf, nbr_fea_idx, crystal_atom_idx,
           crystal_edge_idx, W_rbf, W_full, W_mask, gamma1, beta1, gamma2,
           beta2, res_W1_0, res_b1_0, res_W2_0, res_b2_0, res_W1_1,
           res_b1_1, res_W2_1, res_b2_1):
    idx_dst = nbr_fea_idx[:, 0]
    idx_src = nbr_fea_idx[:, 1]

    a1, a2 = _sc_gather(atom_fea, idx_dst, idx_src)

    ce2 = crystal_edge_idx.reshape(_E, 1)
    tg, stats = _edge_pass1(a1, a2, edge, rbf, ce2, W_rbf, W_full)
    msg = _edge_pass2(tg, ce2, stats, gamma1.reshape(1, -1),
                      beta1.reshape(1, -1), W_mask)

    z128 = jnp.zeros((_N, _AF), jnp.float32)
    acc = _sc_scatter(msg, idx_dst, z128)
    cnt_at = _sc_count(idx_dst, z128)

    out = _final(acc, cnt_at, atom_fea, crystal_atom_idx.reshape(_N, 1),
                 gamma2.reshape(1, -1), beta2.reshape(1, -1),
                 res_W1_0, res_b1_0.reshape(1, -1),
                 res_W2_0, res_b2_0.reshape(1, -1),
                 res_W1_1, res_b1_1.reshape(1, -1),
                 res_W2_1, res_b2_1.reshape(1, -1))
    return out
